# baseline (speedup 1.0000x reference)
import functools

import jax
import jax.numpy as jnp
from jax import lax
from jax.experimental import pallas as pl
from jax.experimental.pallas import tpu as pltpu
from jax.experimental.pallas import tpu_sc as plsc

_NC = 2
_NS = 16
_NW = _NC * _NS
_CL = 128

import numpy as np

_NEG = np.float32(-9e15)
_F32 = jnp.float32
_HI = jax.lax.Precision.HIGHEST


def _dot(a, b):
    return jax.lax.dot_general(a, b, (((1,), (0,)), ((), ())),
                               preferred_element_type=jnp.float32,
                               precision=_HI)


def _leaky(x):
    return jnp.where(x >= 0, x, jnp.float32(0.2) * x)


def _prep_layer(x, W, a_src, a_dst, f_pad, interpret=False):
    n, _ = x.shape
    f = W.shape[1]
    bm = 1000 if n % 1000 == 0 else n

    def body(x_ref, w_ref, as_ref, ad_ref, h_ref, s_ref, d_ref):
        h = _dot(x_ref[...], w_ref[...])
        s_ref[...] = _dot(h, as_ref[...])
        d_ref[...] = _dot(h, ad_ref[...])
        if f_pad > f:
            h = jnp.concatenate(
                [h, jnp.zeros((h.shape[0], f_pad - f), _F32)], axis=1)
        h_ref[...] = h

    h, s, d = pl.pallas_call(
        body,
        grid=(n // bm,),
        in_specs=[
            pl.BlockSpec((bm, x.shape[1]), lambda i: (i, 0)),
            pl.BlockSpec((W.shape[0], f), lambda i: (0, 0)),
            pl.BlockSpec((f, 1), lambda i: (0, 0)),
            pl.BlockSpec((f, 1), lambda i: (0, 0)),
        ],
        out_specs=[
            pl.BlockSpec((bm, f_pad), lambda i: (i, 0)),
            pl.BlockSpec((bm, 1), lambda i: (i, 0)),
            pl.BlockSpec((bm, 1), lambda i: (i, 0)),
        ],
        out_shape=[
            jax.ShapeDtypeStruct((n, f_pad), _F32),
            jax.ShapeDtypeStruct((n, 1), _F32),
            jax.ShapeDtypeStruct((n, 1), _F32),
        ],
        interpret=interpret,
    )(x, W, a_src.reshape(-1, 1), a_dst.reshape(-1, 1))
    return h, s, d


def _edge_prep(edge_feats, a_e0, a_e1, rows, cols, n, interpret=False):
    e, k = edge_feats.shape
    be = 1000 if e % 1000 == 0 else e

    def body(ef_ref, a0_ref, a1_ref, r_ref, c_ref, o0_ref, o1_ref, cell_ref):
        o0_ref[...] = _dot(ef_ref[...], a0_ref[...])
        o1_ref[...] = _dot(ef_ref[...], a1_ref[...])
        cell_ref[...] = r_ref[...] * np.int32(n) + c_ref[...]

    ee0, ee1, cell = pl.pallas_call(
        body,
        grid=(e // be,),
        in_specs=[
            pl.BlockSpec((be, k), lambda i: (i, 0)),
            pl.BlockSpec((k, 1), lambda i: (0, 0)),
            pl.BlockSpec((k, 1), lambda i: (0, 0)),
            pl.BlockSpec((be, 1), lambda i: (i, 0)),
            pl.BlockSpec((be, 1), lambda i: (i, 0)),
        ],
        out_specs=[
            pl.BlockSpec((be, 1), lambda i: (i, 0)),
            pl.BlockSpec((be, 1), lambda i: (i, 0)),
            pl.BlockSpec((be, 1), lambda i: (i, 0)),
        ],
        out_shape=[
            jax.ShapeDtypeStruct((e, 1), _F32),
            jax.ShapeDtypeStruct((e, 1), _F32),
            jax.ShapeDtypeStruct((e, 1), jnp.int32),
        ],
        interpret=interpret,
    )(edge_feats, a_e0.reshape(-1, 1), a_e1.reshape(-1, 1),
      rows.reshape(-1, 1), cols.reshape(-1, 1))
    return ee0[:, 0], ee1[:, 0], cell[:, 0]


def _dense_pass(s, d, adj, h, bm, interpret=False):
    n = adj.shape[0]
    f = h.shape[1]

    def body(s_ref, d_ref, adj_ref, h_ref, m_ref, den_ref, num_ref):
        a = s_ref[...] + d_ref[...]
        e0 = _leaky(a)
        masked = jnp.where(adj_ref[...] > 0, e0, _NEG)
        m = jnp.max(masked, axis=1, keepdims=True)
        p = jnp.exp(masked - m)
        m_ref[...] = m
        den_ref[...] = jnp.sum(p, axis=1, keepdims=True)
        num_ref[...] = _dot(p, h_ref[...])

    m, den, num = pl.pallas_call(
        body,
        grid=(n // bm,),
        in_specs=[
            pl.BlockSpec((bm, 1), lambda i: (i, 0)),
            pl.BlockSpec((1, n), lambda i: (0, 0)),
            pl.BlockSpec((bm, n), lambda i: (i, 0)),
            pl.BlockSpec((n, f), lambda i: (0, 0)),
        ],
        out_specs=[
            pl.BlockSpec((bm, 1), lambda i: (i, 0)),
            pl.BlockSpec((bm, 1), lambda i: (i, 0)),
            pl.BlockSpec((bm, f), lambda i: (i, 0)),
        ],
        out_shape=[
            jax.ShapeDtypeStruct((n, 1), _F32),
            jax.ShapeDtypeStruct((n, 1), _F32),
            jax.ShapeDtypeStruct((n, f), _F32),
        ],
        interpret=interpret,
    )(s, d.reshape(1, -1), adj, h)
    return m, den, num


def _final_combine(num, dnum, den, dden, f_out, interpret=False):
    n = num.shape[0]
    bm = 1000 if n % 1000 == 0 else n

    def body(num_ref, dnum_ref, den_ref, dden_ref, o_ref):
        x = ((num_ref[...][:, :f_out] + dnum_ref[...][:, :f_out])
             / (den_ref[...] + dden_ref[...]))
        o_ref[...] = jnp.where(x > 0, x, jnp.exp(x) - np.float32(1.0))

    return pl.pallas_call(
        body,
        grid=(n // bm,),
        in_specs=[
            pl.BlockSpec((bm, num.shape[1]), lambda i: (i, 0)),
            pl.BlockSpec((bm, dnum.shape[1]), lambda i: (i, 0)),
            pl.BlockSpec((bm, 1), lambda i: (i, 0)),
            pl.BlockSpec((bm, 1), lambda i: (i, 0)),
        ],
        out_specs=pl.BlockSpec((bm, f_out), lambda i: (i, 0)),
        out_shape=jax.ShapeDtypeStruct((n, f_out), _F32),
        interpret=interpret,
    )(num, dnum, den, dden.reshape(-1, 1) if dden.ndim == 1 else dden)


def _combine_prep(num, dnum, den, dden, W, a_src, a_dst, f_pad,
                  interpret=False):
    n = num.shape[0]
    f_in = W.shape[0]
    f = W.shape[1]
    bm = 1000 if n % 1000 == 0 else n

    def body(num_ref, dnum_ref, den_ref, dden_ref, w_ref, as_ref, ad_ref,
             h_ref, s_ref, d_ref):
        x = ((num_ref[...][:, :f_in] + dnum_ref[...][:, :f_in])
             / (den_ref[...] + dden_ref[...]))
        h = _dot(x, w_ref[...])
        s_ref[...] = _dot(h, as_ref[...])
        d_ref[...] = _dot(h, ad_ref[...])
        if f_pad > f:
            h = jnp.concatenate(
                [h, jnp.zeros((h.shape[0], f_pad - f), _F32)], axis=1)
        h_ref[...] = h

    h, s, d = pl.pallas_call(
        body,
        grid=(n // bm,),
        in_specs=[
            pl.BlockSpec((bm, num.shape[1]), lambda i: (i, 0)),
            pl.BlockSpec((bm, dnum.shape[1]), lambda i: (i, 0)),
            pl.BlockSpec((bm, 1), lambda i: (i, 0)),
            pl.BlockSpec((bm, 1), lambda i: (i, 0)),
            pl.BlockSpec((f_in, f), lambda i: (0, 0)),
            pl.BlockSpec((f, 1), lambda i: (0, 0)),
            pl.BlockSpec((f, 1), lambda i: (0, 0)),
        ],
        out_specs=[
            pl.BlockSpec((bm, f_pad), lambda i: (i, 0)),
            pl.BlockSpec((bm, 1), lambda i: (i, 0)),
            pl.BlockSpec((bm, 1), lambda i: (i, 0)),
        ],
        out_shape=[
            jax.ShapeDtypeStruct((n, f_pad), _F32),
            jax.ShapeDtypeStruct((n, 1), _F32),
            jax.ShapeDtypeStruct((n, 1), _F32),
        ],
        interpret=interpret,
    )(num, dnum, den,
      dden.reshape(-1, 1) if dden.ndim == 1 else dden,
      W, a_src.reshape(-1, 1), a_dst.reshape(-1, 1))
    return h, s, d


def _sc_adj_gather(cell3, adj2d, interpret=False):
    nw, nch, _ = cell3.shape
    mesh = plsc.VectorSubcoreMesh(core_axis_name="c", subcore_axis_name="s")
    out_type = [jax.ShapeDtypeStruct((nw, nch, _CL), jnp.int32)]
    scratch = [
        pltpu.VMEM((nch, _CL), jnp.int32),
        pltpu.VMEM((nch, _CL), jnp.int32),
        pltpu.VMEM((nch, _CL), jnp.int32),
        pltpu.VMEM((_CL, _CL), jnp.int32),
        pltpu.VMEM((_CL, _CL), jnp.int32),
        pltpu.SemaphoreType.DMA,
        pltpu.SemaphoreType.DMA,
    ]

    def body(cell_hbm, adj_hbm, adjv_out, cell_v, rowid_v, adjv_v, bufA,
             bufB, semA, semB):
        cid = lax.axis_index("c")
        sid = lax.axis_index("s")
        wid = sid * _NC + cid
        pltpu.sync_copy(cell_hbm.at[wid], cell_v)
        iota = lax.iota(jnp.int32, 16)
        for j in range(nch):
            for g in range(8):
                sl = pl.ds(g * 16, 16)
                rowid_v[j, sl] = lax.shift_right_logical(
                    cell_v[j, sl], jnp.int32(7))

        def process(j, buf):
            for g in range(8):
                sl = pl.ds(g * 16, 16)
                lane = lax.bitwise_and(cell_v[j, sl], jnp.int32(127))
                q16 = iota + g * 16
                adjv_v[j, sl] = plsc.load_gather(buf, [q16, lane])

        pltpu.async_copy(adj_hbm.at[rowid_v.at[0]], bufA, semA).wait()

        def loop_body(it, carry):
            j = it * 2
            pltpu.async_copy(adj_hbm.at[rowid_v.at[j + 1]], bufB, semB)
            process(j, bufA)
            nxt = jnp.where(j + 2 < nch, j + 2, 0)

            @pl.when(j + 2 < nch)
            def _():
                pltpu.async_copy(adj_hbm.at[rowid_v.at[nxt]], bufA, semA)

            pltpu.make_async_copy(adj_hbm.at[rowid_v.at[0]], bufB, semB).wait()
            process(j + 1, bufB)

            @pl.when(j + 2 < nch)
            def _():
                pltpu.make_async_copy(
                    adj_hbm.at[rowid_v.at[0]], bufA, semA).wait()
            return carry

        lax.fori_loop(0, nch // 2, loop_body, 0)
        pltpu.sync_copy(adjv_v, adjv_out.at[wid])

    fn = pl.kernel(body, out_type=out_type, mesh=mesh,
                   scratch_types=scratch, interpret=interpret,
                   compiler_params=pltpu.CompilerParams(
                       needs_layout_passes=False))
    (adjv3,) = fn(cell3, adj2d)
    return adjv3


def _sc_corrections(rows3, cols3, b3, adjv3, s, d, m, h, fr,
                    interpret=False):
    n = s.shape[0]
    f2 = fr + 16
    nw, nch, _ = rows3.shape
    sb = 16
    assert nw == _NW and nch % sb == 0 and h.shape[1] == 128
    nsb = nch // sb
    kz = -(-n // (_NS * _CL))
    n_pad = _NS * kz * _CL

    mesh = plsc.VectorSubcoreMesh(core_axis_name="c", subcore_axis_name="s")
    out_type = [jax.ShapeDtypeStruct((_NC, n_pad, f2), _F32)]
    scratch = [
        pltpu.VMEM((sb, _CL), jnp.int32),
        pltpu.VMEM((sb, _CL), jnp.int32),
        pltpu.VMEM((sb, _CL), _F32),
        pltpu.VMEM((sb, _CL), jnp.int32),
        pltpu.VMEM((n,), _F32),
        pltpu.VMEM((n,), _F32),
        pltpu.VMEM((n,), _F32),
        pltpu.VMEM((_CL, 128), _F32),
        pltpu.VMEM((_CL,), _F32),
        pltpu.VMEM((_CL, f2), _F32),
        pltpu.VMEM((_CL,), jnp.int32),
        pltpu.VMEM_SHARED((n, f2), _F32),
        pltpu.SemaphoreType.DMA,
    ]

    def body(rows_hbm, cols_hbm, b_hbm, adjv_hbm, s_hbm, d_hbm, m_hbm, h_hbm,
             out_hbm, rows_v, cols_v, b_v, adjv_v, s_v, d_v, m_v,
             hrA, delta_v, vals_v, idx_v, acc, semA):
        cid = lax.axis_index("c")
        sid = lax.axis_index("s")
        wid = sid * _NC + cid
        iota = lax.iota(jnp.int32, 16)

        pltpu.sync_copy(s_hbm, s_v)
        pltpu.sync_copy(d_hbm, d_v)
        pltpu.sync_copy(m_hbm, m_v)

        def build_idx(base):
            for g in range(8):
                idx_v[pl.ds(g * 16, 16)] = jnp.minimum(
                    iota + (base + g * 16), np.int32(n - 1))

        zero16 = jnp.zeros((16,), _F32)
        for r in range(_CL):
            for f0 in range(0, f2, 16):
                vals_v[r, pl.ds(f0, 16)] = zero16
        for k in range(kz):
            base = (sid * kz + k) * _CL
            build_idx(base)
            pltpu.sync_copy(vals_v, acc.at[idx_v])
        plsc.subcore_barrier()

        unit = jnp.where(iota == 0, np.float32(1.0), np.float32(0.0))

        def process(j, hr):
            for g in range(8):
                sl = pl.ds(g * 16, 16)
                r16 = rows_v[j, sl]
                c16 = cols_v[j, sl]
                b16 = b_v[j, sl]
                a16 = adjv_v[j, sl]
                si = plsc.load_gather(s_v, [r16])
                dj = plsc.load_gather(d_v, [c16])
                mi = plsc.load_gather(m_v, [r16])
                a = si + dj
                p0 = jnp.exp(_leaky(a) - mi)
                p1 = jnp.exp(_leaky(a + b16) - mi)
                delta = jnp.where(a16 > 0, p1 - p0, np.float32(0.0))
                delta_v[sl] = delta
            for q in range(_CL):
                dq = plsc.load_gather(delta_v, [iota * 0 + q])
                for f0 in range(0, fr, 16):
                    vals_v[q, pl.ds(f0, 16)] = dq * hr[q, pl.ds(f0, 16)]
                vals_v[q, pl.ds(fr, 16)] = dq * unit
            pltpu.sync_copy(vals_v, acc.at[rows_v.at[j]], add=True)

        def sblock(u, carry):
            off = pl.multiple_of(u * sb, 8)
            usl = pl.ds(off, sb)
            pltpu.sync_copy(rows_hbm.at[wid].at[usl], rows_v)
            pltpu.sync_copy(cols_hbm.at[wid].at[usl], cols_v)
            pltpu.sync_copy(b_hbm.at[wid].at[usl], b_v)
            pltpu.sync_copy(adjv_hbm.at[wid].at[usl], adjv_v)

            def loop_body(j, carry2):
                pltpu.async_copy(h_hbm.at[cols_v.at[j]], hrA, semA).wait()
                process(j, hrA)
                return carry2

            lax.fori_loop(0, sb, loop_body, 0)
            return carry

        lax.fori_loop(0, nsb, sblock, 0)

        plsc.subcore_barrier()
        for k in range(kz):
            c = sid * kz + k
            base = c * _CL
            build_idx(base)
            pltpu.sync_copy(acc.at[idx_v], vals_v)
            obase = pl.multiple_of(c * _CL, _CL)
            pltpu.sync_copy(vals_v, out_hbm.at[cid].at[pl.ds(obase, _CL)])

    fn = pl.kernel(body, out_type=out_type, mesh=mesh,
                   scratch_types=scratch, interpret=interpret,
                   compiler_params=pltpu.CompilerParams(
                       needs_layout_passes=False))
    (acc_out,) = fn(rows3, cols3, b3, adjv3, s, d, m, h)
    return acc_out[:, :n, :]


def _run(node_feats, edge_feats, edge_indices, adj, W0, a_src0, a_dst0, a_e0,
         W1, a_src1, a_dst1, a_e1, interpret=False):
    n = node_feats.shape[0]
    e = edge_feats.shape[0]
    hid = W0.shape[1]
    ncls = W1.shape[1]
    fr0 = hid + (-hid) % 16
    fr1 = ncls + (-ncls) % 16
    fpad = 128
    bm = 40 if n % 40 == 0 else n

    rows = edge_indices[0].astype(jnp.int32)
    cols = edge_indices[1].astype(jnp.int32)
    ee0, ee1, cell = _edge_prep(edge_feats, a_e0, a_e1, rows, cols, n,
                                interpret=interpret)
    ept = -(-e // _NW)
    nch = -(-ept // _CL)
    nch += (-nch) % 16
    ep = _NW * nch * _CL

    def to3(x):
        return jnp.pad(x, (0, ep - e)).reshape(_NW, nch, _CL)

    rows3, cols3, cell3 = to3(rows), to3(cols), to3(cell)
    b03, b13 = to3(ee0), to3(ee1)
    adj2d = adj.reshape(-1, 128).astype(jnp.int32)
    adjv3 = _sc_adj_gather(cell3, adj2d, interpret=interpret)

    h0, s0, d0 = _prep_layer(node_feats, W0, a_src0, a_dst0, fpad,
                             interpret=interpret)
    m0, den0, num0 = _dense_pass(s0, d0[:, 0], adj, h0, bm,
                                 interpret=interpret)
    acc0 = _sc_corrections(rows3, cols3, b03, adjv3, s0[:, 0], d0[:, 0],
                           m0[:, 0], h0, fr0, interpret=interpret)
    accs0 = acc0[0] + acc0[1]
    dden0 = accs0[:, fr0]

    h1, s1, d1 = _combine_prep(num0, accs0, den0, dden0, W1, a_src1, a_dst1,
                               fpad, interpret=interpret)
    m1, den1, num1 = _dense_pass(s1, d1[:, 0], adj, h1, bm,
                                 interpret=interpret)
    acc1 = _sc_corrections(rows3, cols3, b13, adjv3, s1[:, 0], d1[:, 0],
                           m1[:, 0], h1, fr1, interpret=interpret)
    accs1 = acc1[0] + acc1[1]
    dden1 = accs1[:, fr1]

    return _final_combine(num1, accs1, den1, dden1, ncls,
                          interpret=interpret)


def kernel(node_feats, edge_feats, edge_indices, adj, W0, a_src0, a_dst0,
           a_e0, W1, a_src1, a_dst1, a_e1):
    return _run(node_feats, edge_feats, edge_indices, adj, W0, a_src0,
                a_dst0, a_e0, W1, a_src1, a_dst1, a_e1)


# speedup vs baseline: 1.0372x; 1.0372x over previous
import functools

import jax
import jax.numpy as jnp
from jax import lax
from jax.experimental import pallas as pl
from jax.experimental.pallas import tpu as pltpu
from jax.experimental.pallas import tpu_sc as plsc

_NC = 2
_NS = 16
_NW = _NC * _NS
_CL = 128

import numpy as np

_NEG = np.float32(-9e15)
_F32 = jnp.float32
_HI = jax.lax.Precision.HIGHEST


def _dot(a, b):
    return jax.lax.dot_general(a, b, (((1,), (0,)), ((), ())),
                               preferred_element_type=jnp.float32,
                               precision=_HI)


def _leaky(x):
    return jnp.where(x >= 0, x, jnp.float32(0.2) * x)


def _prep_layer(x, W, a_src, a_dst, f_pad, interpret=False):
    n, _ = x.shape
    f = W.shape[1]
    bm = 1000 if n % 1000 == 0 else n

    def body(x_ref, w_ref, as_ref, ad_ref, h_ref, s_ref, d_ref):
        h = _dot(x_ref[...], w_ref[...])
        s_ref[...] = _dot(h, as_ref[...])
        d_ref[...] = _dot(h, ad_ref[...])
        if f_pad > f:
            h = jnp.concatenate(
                [h, jnp.zeros((h.shape[0], f_pad - f), _F32)], axis=1)
        h_ref[...] = h

    h, s, d = pl.pallas_call(
        body,
        grid=(n // bm,),
        in_specs=[
            pl.BlockSpec((bm, x.shape[1]), lambda i: (i, 0)),
            pl.BlockSpec((W.shape[0], f), lambda i: (0, 0)),
            pl.BlockSpec((f, 1), lambda i: (0, 0)),
            pl.BlockSpec((f, 1), lambda i: (0, 0)),
        ],
        out_specs=[
            pl.BlockSpec((bm, f_pad), lambda i: (i, 0)),
            pl.BlockSpec((bm, 1), lambda i: (i, 0)),
            pl.BlockSpec((bm, 1), lambda i: (i, 0)),
        ],
        out_shape=[
            jax.ShapeDtypeStruct((n, f_pad), _F32),
            jax.ShapeDtypeStruct((n, 1), _F32),
            jax.ShapeDtypeStruct((n, 1), _F32),
        ],
        interpret=interpret,
    )(x, W, a_src.reshape(-1, 1), a_dst.reshape(-1, 1))
    return h, s, d


def _edge_prep(edge_feats, a_e0, a_e1, rows, cols, n, interpret=False):
    e, k = edge_feats.shape
    be = 1000 if e % 1000 == 0 else e

    def body(ef_ref, a0_ref, a1_ref, r_ref, c_ref, o0_ref, o1_ref, cell_ref):
        o0_ref[...] = _dot(ef_ref[...], a0_ref[...])
        o1_ref[...] = _dot(ef_ref[...], a1_ref[...])
        cell_ref[...] = r_ref[...] * np.int32(n) + c_ref[...]

    ee0, ee1, cell = pl.pallas_call(
        body,
        grid=(e // be,),
        in_specs=[
            pl.BlockSpec((be, k), lambda i: (i, 0)),
            pl.BlockSpec((k, 1), lambda i: (0, 0)),
            pl.BlockSpec((k, 1), lambda i: (0, 0)),
            pl.BlockSpec((be, 1), lambda i: (i, 0)),
            pl.BlockSpec((be, 1), lambda i: (i, 0)),
        ],
        out_specs=[
            pl.BlockSpec((be, 1), lambda i: (i, 0)),
            pl.BlockSpec((be, 1), lambda i: (i, 0)),
            pl.BlockSpec((be, 1), lambda i: (i, 0)),
        ],
        out_shape=[
            jax.ShapeDtypeStruct((e, 1), _F32),
            jax.ShapeDtypeStruct((e, 1), _F32),
            jax.ShapeDtypeStruct((e, 1), jnp.int32),
        ],
        interpret=interpret,
    )(edge_feats, a_e0.reshape(-1, 1), a_e1.reshape(-1, 1),
      rows.reshape(-1, 1), cols.reshape(-1, 1))
    return ee0[:, 0], ee1[:, 0], cell[:, 0]


def _dense_pass(s, d, adj, h, bm, interpret=False):
    n = adj.shape[0]
    f = h.shape[1]

    def body(s_ref, d_ref, adj_ref, h_ref, m_ref, den_ref, num_ref):
        a = s_ref[...] + d_ref[...]
        e0 = _leaky(a)
        masked = jnp.where(adj_ref[...] > 0, e0, _NEG)
        m = jnp.max(masked, axis=1, keepdims=True)
        p = jnp.exp(masked - m)
        m_ref[...] = m
        den_ref[...] = jnp.sum(p, axis=1, keepdims=True)
        num_ref[...] = _dot(p, h_ref[...])

    m, den, num = pl.pallas_call(
        body,
        grid=(n // bm,),
        in_specs=[
            pl.BlockSpec((bm, 1), lambda i: (i, 0)),
            pl.BlockSpec((1, n), lambda i: (0, 0)),
            pl.BlockSpec((bm, n), lambda i: (i, 0)),
            pl.BlockSpec((n, f), lambda i: (0, 0)),
        ],
        out_specs=[
            pl.BlockSpec((bm, 1), lambda i: (i, 0)),
            pl.BlockSpec((bm, 1), lambda i: (i, 0)),
            pl.BlockSpec((bm, f), lambda i: (i, 0)),
        ],
        out_shape=[
            jax.ShapeDtypeStruct((n, 1), _F32),
            jax.ShapeDtypeStruct((n, 1), _F32),
            jax.ShapeDtypeStruct((n, f), _F32),
        ],
        interpret=interpret,
    )(s, d.reshape(1, -1), adj, h)
    return m, den, num


def _final_combine(num, dnum, den, dden, f_out, interpret=False):
    n = num.shape[0]
    bm = 1000 if n % 1000 == 0 else n

    def body(num_ref, dnum_ref, den_ref, dden_ref, o_ref):
        x = ((num_ref[...][:, :f_out] + dnum_ref[...][:, :f_out])
             / (den_ref[...] + dden_ref[...]))
        o_ref[...] = jnp.where(x > 0, x, jnp.exp(x) - np.float32(1.0))

    return pl.pallas_call(
        body,
        grid=(n // bm,),
        in_specs=[
            pl.BlockSpec((bm, num.shape[1]), lambda i: (i, 0)),
            pl.BlockSpec((bm, dnum.shape[1]), lambda i: (i, 0)),
            pl.BlockSpec((bm, 1), lambda i: (i, 0)),
            pl.BlockSpec((bm, 1), lambda i: (i, 0)),
        ],
        out_specs=pl.BlockSpec((bm, f_out), lambda i: (i, 0)),
        out_shape=jax.ShapeDtypeStruct((n, f_out), _F32),
        interpret=interpret,
    )(num, dnum, den, dden.reshape(-1, 1) if dden.ndim == 1 else dden)


def _combine_prep(num, dnum, den, dden, W, a_src, a_dst, f_pad,
                  interpret=False):
    n = num.shape[0]
    f_in = W.shape[0]
    f = W.shape[1]
    bm = 1000 if n % 1000 == 0 else n

    def body(num_ref, dnum_ref, den_ref, dden_ref, w_ref, as_ref, ad_ref,
             h_ref, s_ref, d_ref):
        x = ((num_ref[...][:, :f_in] + dnum_ref[...][:, :f_in])
             / (den_ref[...] + dden_ref[...]))
        h = _dot(x, w_ref[...])
        s_ref[...] = _dot(h, as_ref[...])
        d_ref[...] = _dot(h, ad_ref[...])
        if f_pad > f:
            h = jnp.concatenate(
                [h, jnp.zeros((h.shape[0], f_pad - f), _F32)], axis=1)
        h_ref[...] = h

    h, s, d = pl.pallas_call(
        body,
        grid=(n // bm,),
        in_specs=[
            pl.BlockSpec((bm, num.shape[1]), lambda i: (i, 0)),
            pl.BlockSpec((bm, dnum.shape[1]), lambda i: (i, 0)),
            pl.BlockSpec((bm, 1), lambda i: (i, 0)),
            pl.BlockSpec((bm, 1), lambda i: (i, 0)),
            pl.BlockSpec((f_in, f), lambda i: (0, 0)),
            pl.BlockSpec((f, 1), lambda i: (0, 0)),
            pl.BlockSpec((f, 1), lambda i: (0, 0)),
        ],
        out_specs=[
            pl.BlockSpec((bm, f_pad), lambda i: (i, 0)),
            pl.BlockSpec((bm, 1), lambda i: (i, 0)),
            pl.BlockSpec((bm, 1), lambda i: (i, 0)),
        ],
        out_shape=[
            jax.ShapeDtypeStruct((n, f_pad), _F32),
            jax.ShapeDtypeStruct((n, 1), _F32),
            jax.ShapeDtypeStruct((n, 1), _F32),
        ],
        interpret=interpret,
    )(num, dnum, den,
      dden.reshape(-1, 1) if dden.ndim == 1 else dden,
      W, a_src.reshape(-1, 1), a_dst.reshape(-1, 1))
    return h, s, d


def _sc_adj_gather(cell3, adj2d, interpret=False):
    nw, nch, _ = cell3.shape
    mesh = plsc.VectorSubcoreMesh(core_axis_name="c", subcore_axis_name="s")
    out_type = [jax.ShapeDtypeStruct((nw, nch, _CL), jnp.int32)]
    scratch = [
        pltpu.VMEM((nch, _CL), jnp.int32),
        pltpu.VMEM((nch, _CL), jnp.int32),
        pltpu.VMEM((nch, _CL), jnp.int32),
        pltpu.VMEM((4, _CL, _CL), jnp.int32),
        pltpu.SemaphoreType.DMA,
        pltpu.SemaphoreType.DMA,
        pltpu.SemaphoreType.DMA,
        pltpu.SemaphoreType.DMA,
    ]

    def body(cell_hbm, adj_hbm, adjv_out, cell_v, rowid_v, adjv_v, bufs,
             *sems):
        cid = lax.axis_index("c")
        sid = lax.axis_index("s")
        wid = sid * _NC + cid
        pltpu.sync_copy(cell_hbm.at[wid], cell_v)
        iota = lax.iota(jnp.int32, 16)
        for j in range(nch):
            for g in range(8):
                sl = pl.ds(g * 16, 16)
                rowid_v[j, sl] = lax.shift_right_logical(
                    cell_v[j, sl], jnp.int32(7))

        def process(j, r):
            for g in range(8):
                sl = pl.ds(g * 16, 16)
                lane = lax.bitwise_and(cell_v[j, sl], jnp.int32(127))
                q16 = iota + g * 16
                adjv_v[j, sl] = plsc.load_gather(bufs.at[r], [q16, lane])

        assert nch % 4 == 0
        for r in range(4):
            pltpu.async_copy(adj_hbm.at[rowid_v.at[r]], bufs.at[r], sems[r])

        def loop_body(it, carry):
            j0 = it * 4
            for r in range(4):
                j = j0 + r
                pltpu.make_async_copy(
                    adj_hbm.at[rowid_v.at[0]], bufs.at[r], sems[r]).wait()
                process(j, r)

                @pl.when(j + 4 < nch)
                def _():
                    nxt = jnp.where(j + 4 < nch, j + 4, 0)
                    pltpu.async_copy(
                        adj_hbm.at[rowid_v.at[nxt]], bufs.at[r], sems[r])
            return carry

        lax.fori_loop(0, nch // 4, loop_body, 0)
        pltpu.sync_copy(adjv_v, adjv_out.at[wid])

    fn = pl.kernel(body, out_type=out_type, mesh=mesh,
                   scratch_types=scratch, interpret=interpret,
                   compiler_params=pltpu.CompilerParams(
                       needs_layout_passes=False))
    (adjv3,) = fn(cell3, adj2d)
    return adjv3


def _sc_corrections(rows3, cols3, b3, adjv3, s, d, m, h, fr,
                    interpret=False):
    n = s.shape[0]
    f2 = 16 * (-(-(fr + 1) // 16))
    dblk = fr // 16
    nw, nch, _ = rows3.shape
    sb = 8
    assert nw == _NW and nch % sb == 0 and h.shape[1] == 128
    nsb = nch // sb
    kz = -(-n // (_NS * _CL))
    n_pad = _NS * kz * _CL

    mesh = plsc.VectorSubcoreMesh(core_axis_name="c", subcore_axis_name="s")
    out_type = [jax.ShapeDtypeStruct((_NC, n_pad, f2), _F32)]
    scratch = [
        pltpu.VMEM((sb, _CL), jnp.int32),
        pltpu.VMEM((sb, _CL), jnp.int32),
        pltpu.VMEM((sb, _CL), _F32),
        pltpu.VMEM((sb, _CL), jnp.int32),
        pltpu.VMEM((n,), _F32),
        pltpu.VMEM((n,), _F32),
        pltpu.VMEM((n,), _F32),
        pltpu.VMEM((2, _CL, 128), _F32),
        pltpu.VMEM((_CL,), _F32),
        pltpu.VMEM((2, _CL, f2), _F32),
        pltpu.VMEM((_CL,), jnp.int32),
        pltpu.VMEM_SHARED((n, f2), _F32),
        pltpu.SemaphoreType.DMA,
        pltpu.SemaphoreType.DMA,
        pltpu.SemaphoreType.DMA,
        pltpu.SemaphoreType.DMA,
        pltpu.SemaphoreType.DMA,
    ]

    def body(rows_hbm, cols_hbm, b_hbm, adjv_hbm, s_hbm, d_hbm, m_hbm, h_hbm,
             out_hbm, rows_v, cols_v, b_v, adjv_v, s_v, d_v, m_v,
             hrs, delta_v, valss, idx_v, acc, semHA, semHB, semSA, semSB,
             semST):
        cid = lax.axis_index("c")
        sid = lax.axis_index("s")
        wid = sid * _NC + cid
        iota = lax.iota(jnp.int32, 16)

        pltpu.sync_copy(s_hbm, s_v)
        pltpu.sync_copy(d_hbm, d_v)
        pltpu.sync_copy(m_hbm, m_v)

        def build_idx(base):
            for g in range(8):
                idx_v[pl.ds(g * 16, 16)] = jnp.minimum(
                    iota + (base + g * 16), np.int32(n - 1))

        zero16 = jnp.zeros((16,), _F32)
        for r in range(_CL):
            for f0 in range(0, f2, 16):
                valss[0, r, pl.ds(f0, 16)] = zero16
        for k in range(kz):
            base = (sid * kz + k) * _CL
            build_idx(base)
            pltpu.sync_copy(valss.at[0], acc.at[idx_v])
        plsc.subcore_barrier()

        unit = jnp.where(iota == fr % 16, np.float32(1.0), np.float32(0.0))
        semH = (semHA, semHB)
        semS = (semSA, semSB)

        def process(j, p):
            hr = hrs.at[p]
            for g in range(8):
                sl = pl.ds(g * 16, 16)
                r16 = rows_v[j, sl]
                c16 = cols_v[j, sl]
                b16 = b_v[j, sl]
                a16 = adjv_v[j, sl]
                si = plsc.load_gather(s_v, [r16])
                dj = plsc.load_gather(d_v, [c16])
                mi = plsc.load_gather(m_v, [r16])
                a = si + dj
                p0 = jnp.exp(_leaky(a) - mi)
                p1 = jnp.exp(_leaky(a + b16) - mi)
                delta = jnp.where(a16 > 0, p1 - p0, np.float32(0.0))
                delta_v[sl] = delta
            for q in range(_CL):
                dq = plsc.load_gather(delta_v, [iota * 0 + q])
                for blk in range(f2 // 16):
                    hv = hr[q, pl.ds(blk * 16, 16)]
                    if blk == dblk:
                        hv = hv + unit
                    valss[p, q, pl.ds(blk * 16, 16)] = dq * hv
            pltpu.async_copy(valss.at[p], acc.at[rows_v.at[j]], semS[p],
                             add=True)

        def drain_scatter(p):
            pltpu.make_async_copy(
                valss.at[p], acc.at[rows_v.at[0]], semS[p]).wait()

        def fire_h(j, p):
            pltpu.async_copy(h_hbm.at[cols_v.at[j]], hrs.at[p], semH[p])

        def wait_h(p):
            pltpu.make_async_copy(
                h_hbm.at[cols_v.at[0]], hrs.at[p], semH[p]).wait()

        def sblock(u, carry):
            off = pl.multiple_of(u * sb, 8)
            usl = pl.ds(off, sb)
            cps = [pltpu.async_copy(rows_hbm.at[wid].at[usl], rows_v, semST),
                   pltpu.async_copy(cols_hbm.at[wid].at[usl], cols_v, semST),
                   pltpu.async_copy(b_hbm.at[wid].at[usl], b_v, semST),
                   pltpu.async_copy(adjv_hbm.at[wid].at[usl], adjv_v, semST)]
            for cp in cps:
                cp.wait()
            fire_h(0, 0)

            def loop_body(it, carry2):
                j = it * 2
                fire_h(j + 1, 1)
                wait_h(0)

                @pl.when(it > 0)
                def _():
                    drain_scatter(0)
                process(j, 0)

                @pl.when(j + 2 < sb)
                def _():
                    nxt = jnp.where(j + 2 < sb, j + 2, 0)
                    fire_h(nxt, 0)
                wait_h(1)

                @pl.when(it > 0)
                def _():
                    drain_scatter(1)
                process(j + 1, 1)
                return carry2

            lax.fori_loop(0, sb // 2, loop_body, 0)
            drain_scatter(0)
            drain_scatter(1)
            return carry

        lax.fori_loop(0, nsb, sblock, 0)

        plsc.subcore_barrier()
        for k in range(kz):
            c = sid * kz + k
            base = c * _CL
            build_idx(base)
            pltpu.sync_copy(acc.at[idx_v], valss.at[0])
            obase = pl.multiple_of(c * _CL, _CL)
            pltpu.sync_copy(valss.at[0], out_hbm.at[cid].at[pl.ds(obase, _CL)])

    fn = pl.kernel(body, out_type=out_type, mesh=mesh,
                   scratch_types=scratch, interpret=interpret,
                   compiler_params=pltpu.CompilerParams(
                       needs_layout_passes=False))
    (acc_out,) = fn(rows3, cols3, b3, adjv3, s, d, m, h)
    return acc_out[:, :n, :]


def _run(node_feats, edge_feats, edge_indices, adj, W0, a_src0, a_dst0, a_e0,
         W1, a_src1, a_dst1, a_e1, interpret=False):
    n = node_feats.shape[0]
    e = edge_feats.shape[0]
    hid = W0.shape[1]
    ncls = W1.shape[1]
    fr0 = hid
    fr1 = ncls
    fpad = 128
    bm = 40 if n % 40 == 0 else n

    rows = edge_indices[0].astype(jnp.int32)
    cols = edge_indices[1].astype(jnp.int32)
    ee0, ee1, cell = _edge_prep(edge_feats, a_e0, a_e1, rows, cols, n,
                                interpret=interpret)
    ept = -(-e // _NW)
    nch = -(-ept // _CL)
    nch += (-nch) % 16
    ep = _NW * nch * _CL

    def to3(x):
        return jnp.pad(x, (0, ep - e)).reshape(_NW, nch, _CL)

    rows3, cols3, cell3 = to3(rows), to3(cols), to3(cell)
    b03, b13 = to3(ee0), to3(ee1)
    adj2d = adj.reshape(-1, 128).astype(jnp.int32)
    adjv3 = _sc_adj_gather(cell3, adj2d, interpret=interpret)

    h0, s0, d0 = _prep_layer(node_feats, W0, a_src0, a_dst0, fpad,
                             interpret=interpret)
    m0, den0, num0 = _dense_pass(s0, d0[:, 0], adj, h0, bm,
                                 interpret=interpret)
    acc0 = _sc_corrections(rows3, cols3, b03, adjv3, s0[:, 0], d0[:, 0],
                           m0[:, 0], h0, fr0, interpret=interpret)
    accs0 = acc0[0] + acc0[1]
    dden0 = accs0[:, fr0]

    h1, s1, d1 = _combine_prep(num0, accs0, den0, dden0, W1, a_src1, a_dst1,
                               fpad, interpret=interpret)
    m1, den1, num1 = _dense_pass(s1, d1[:, 0], adj, h1, bm,
                                 interpret=interpret)
    acc1 = _sc_corrections(rows3, cols3, b13, adjv3, s1[:, 0], d1[:, 0],
                           m1[:, 0], h1, fr1, interpret=interpret)
    accs1 = acc1[0] + acc1[1]
    dden1 = accs1[:, fr1]

    return _final_combine(num1, accs1, den1, dden1, ncls,
                          interpret=interpret)


def kernel(node_feats, edge_feats, edge_indices, adj, W0, a_src0, a_dst0,
           a_e0, W1, a_src1, a_dst1, a_e1):
    return _run(node_feats, edge_feats, edge_indices, adj, W0, a_src0,
                a_dst0, a_e0, W1, a_src1, a_dst1, a_e1)


# speedup vs baseline: 1.1550x; 1.1136x over previous
import functools

import jax
import jax.numpy as jnp
from jax import lax
from jax.experimental import pallas as pl
from jax.experimental.pallas import tpu as pltpu
from jax.experimental.pallas import tpu_sc as plsc

_NC = 2
_NS = 16
_NW = _NC * _NS
_CL = 128

import numpy as np

_NEG = np.float32(-9e15)
_F32 = jnp.float32
_HI = jax.lax.Precision.HIGHEST


def _dot(a, b):
    return jax.lax.dot_general(a, b, (((1,), (0,)), ((), ())),
                               preferred_element_type=jnp.float32,
                               precision=_HI)


def _leaky(x):
    return jnp.where(x >= 0, x, jnp.float32(0.2) * x)


def _prep_layer(x, W, a_src, a_dst, f_pad, interpret=False):
    n, _ = x.shape
    f = W.shape[1]
    bm = 1000 if n % 1000 == 0 else n

    def body(x_ref, w_ref, as_ref, ad_ref, h_ref, hp_ref, s_ref, d_ref):
        h = _dot(x_ref[...], w_ref[...])
        s_ref[...] = _dot(h, as_ref[...])
        d_ref[...] = _dot(h, ad_ref[...])
        h_ref[...] = h
        hp_ref[...] = jnp.concatenate(
            [h, jnp.zeros((h.shape[0], f_pad - f), _F32)], axis=1)

    h, hp, s, d = pl.pallas_call(
        body,
        grid=(n // bm,),
        in_specs=[
            pl.BlockSpec((bm, x.shape[1]), lambda i: (i, 0)),
            pl.BlockSpec((W.shape[0], f), lambda i: (0, 0)),
            pl.BlockSpec((f, 1), lambda i: (0, 0)),
            pl.BlockSpec((f, 1), lambda i: (0, 0)),
        ],
        out_specs=[
            pl.BlockSpec((bm, f), lambda i: (i, 0)),
            pl.BlockSpec((bm, f_pad), lambda i: (i, 0)),
            pl.BlockSpec((bm, 1), lambda i: (i, 0)),
            pl.BlockSpec((bm, 1), lambda i: (i, 0)),
        ],
        out_shape=[
            jax.ShapeDtypeStruct((n, f), _F32),
            jax.ShapeDtypeStruct((n, f_pad), _F32),
            jax.ShapeDtypeStruct((n, 1), _F32),
            jax.ShapeDtypeStruct((n, 1), _F32),
        ],
        interpret=interpret,
    )(x, W, a_src.reshape(-1, 1), a_dst.reshape(-1, 1))
    return h, hp, s, d


def _edge_prep(edge_feats, a_e0, a_e1, rows, cols, n, interpret=False):
    e, k = edge_feats.shape
    be = 1000 if e % 1000 == 0 else e

    def body(ef_ref, a0_ref, a1_ref, r_ref, c_ref, o0_ref, o1_ref, cell_ref):
        o0_ref[...] = _dot(ef_ref[...], a0_ref[...])
        o1_ref[...] = _dot(ef_ref[...], a1_ref[...])
        cell_ref[...] = r_ref[...] * np.int32(n) + c_ref[...]

    ee0, ee1, cell = pl.pallas_call(
        body,
        grid=(e // be,),
        in_specs=[
            pl.BlockSpec((be, k), lambda i: (i, 0)),
            pl.BlockSpec((k, 1), lambda i: (0, 0)),
            pl.BlockSpec((k, 1), lambda i: (0, 0)),
            pl.BlockSpec((be, 1), lambda i: (i, 0)),
            pl.BlockSpec((be, 1), lambda i: (i, 0)),
        ],
        out_specs=[
            pl.BlockSpec((be, 1), lambda i: (i, 0)),
            pl.BlockSpec((be, 1), lambda i: (i, 0)),
            pl.BlockSpec((be, 1), lambda i: (i, 0)),
        ],
        out_shape=[
            jax.ShapeDtypeStruct((e, 1), _F32),
            jax.ShapeDtypeStruct((e, 1), _F32),
            jax.ShapeDtypeStruct((e, 1), jnp.int32),
        ],
        interpret=interpret,
    )(edge_feats, a_e0.reshape(-1, 1), a_e1.reshape(-1, 1),
      rows.reshape(-1, 1), cols.reshape(-1, 1))
    return ee0[:, 0], ee1[:, 0], cell[:, 0]


def _dense_pass(s, d, adj, h, bm, interpret=False):
    n = adj.shape[0]
    f = h.shape[1]

    def body(s_ref, d_ref, adj_ref, h_ref, m_ref, den_ref, num_ref):
        a = s_ref[...] + d_ref[...]
        e0 = _leaky(a)
        masked = jnp.where(adj_ref[...] > 0, e0, _NEG)
        m = jnp.max(masked, axis=1, keepdims=True)
        p = jnp.exp(masked - m)
        m_ref[...] = m
        den_ref[...] = jnp.sum(p, axis=1, keepdims=True)
        num_ref[...] = _dot(p, h_ref[...])

    m, den, num = pl.pallas_call(
        body,
        grid=(n // bm,),
        in_specs=[
            pl.BlockSpec((bm, 1), lambda i: (i, 0)),
            pl.BlockSpec((1, n), lambda i: (0, 0)),
            pl.BlockSpec((bm, n), lambda i: (i, 0)),
            pl.BlockSpec((n, f), lambda i: (0, 0)),
        ],
        out_specs=[
            pl.BlockSpec((bm, 1), lambda i: (i, 0)),
            pl.BlockSpec((bm, 1), lambda i: (i, 0)),
            pl.BlockSpec((bm, f), lambda i: (i, 0)),
        ],
        out_shape=[
            jax.ShapeDtypeStruct((n, 1), _F32),
            jax.ShapeDtypeStruct((n, 1), _F32),
            jax.ShapeDtypeStruct((n, f), _F32),
        ],
        interpret=interpret,
    )(s, d.reshape(1, -1), adj, h)
    return m, den, num


def _final_combine(num, dnum, den, dden, f_out, interpret=False):
    n = num.shape[0]
    bm = 1000 if n % 1000 == 0 else n

    def body(num_ref, dnum_ref, den_ref, dden_ref, o_ref):
        x = ((num_ref[...][:, :f_out] + dnum_ref[...][:, :f_out])
             / (den_ref[...] + dden_ref[...]))
        o_ref[...] = jnp.where(x > 0, x, jnp.exp(x) - np.float32(1.0))

    return pl.pallas_call(
        body,
        grid=(n // bm,),
        in_specs=[
            pl.BlockSpec((bm, num.shape[1]), lambda i: (i, 0)),
            pl.BlockSpec((bm, dnum.shape[1]), lambda i: (i, 0)),
            pl.BlockSpec((bm, 1), lambda i: (i, 0)),
            pl.BlockSpec((bm, 1), lambda i: (i, 0)),
        ],
        out_specs=pl.BlockSpec((bm, f_out), lambda i: (i, 0)),
        out_shape=jax.ShapeDtypeStruct((n, f_out), _F32),
        interpret=interpret,
    )(num, dnum, den, dden.reshape(-1, 1) if dden.ndim == 1 else dden)


def _combine_prep(num, dnum, den, dden, W, a_src, a_dst, f_pad,
                  interpret=False):
    n = num.shape[0]
    f_in = W.shape[0]
    f = W.shape[1]
    bm = 1000 if n % 1000 == 0 else n

    def body(num_ref, dnum_ref, den_ref, dden_ref, w_ref, as_ref, ad_ref,
             h_ref, hp_ref, s_ref, d_ref):
        x = ((num_ref[...][:, :f_in] + dnum_ref[...][:, :f_in])
             / (den_ref[...] + dden_ref[...]))
        h = _dot(x, w_ref[...])
        s_ref[...] = _dot(h, as_ref[...])
        d_ref[...] = _dot(h, ad_ref[...])
        h_ref[...] = h
        hp_ref[...] = jnp.concatenate(
            [h, jnp.zeros((h.shape[0], f_pad - f), _F32)], axis=1)

    h, hp, s, d = pl.pallas_call(
        body,
        grid=(n // bm,),
        in_specs=[
            pl.BlockSpec((bm, num.shape[1]), lambda i: (i, 0)),
            pl.BlockSpec((bm, dnum.shape[1]), lambda i: (i, 0)),
            pl.BlockSpec((bm, 1), lambda i: (i, 0)),
            pl.BlockSpec((bm, 1), lambda i: (i, 0)),
            pl.BlockSpec((f_in, f), lambda i: (0, 0)),
            pl.BlockSpec((f, 1), lambda i: (0, 0)),
            pl.BlockSpec((f, 1), lambda i: (0, 0)),
        ],
        out_specs=[
            pl.BlockSpec((bm, f), lambda i: (i, 0)),
            pl.BlockSpec((bm, f_pad), lambda i: (i, 0)),
            pl.BlockSpec((bm, 1), lambda i: (i, 0)),
            pl.BlockSpec((bm, 1), lambda i: (i, 0)),
        ],
        out_shape=[
            jax.ShapeDtypeStruct((n, f), _F32),
            jax.ShapeDtypeStruct((n, f_pad), _F32),
            jax.ShapeDtypeStruct((n, 1), _F32),
            jax.ShapeDtypeStruct((n, 1), _F32),
        ],
        interpret=interpret,
    )(num, dnum, den,
      dden.reshape(-1, 1) if dden.ndim == 1 else dden,
      W, a_src.reshape(-1, 1), a_dst.reshape(-1, 1))
    return h, hp, s, d


def _sc_adj_gather(cell3, adj2d, interpret=False):
    nw, nch, _ = cell3.shape
    mesh = plsc.VectorSubcoreMesh(core_axis_name="c", subcore_axis_name="s")
    out_type = [jax.ShapeDtypeStruct((nw, nch, _CL), jnp.int32)]
    scratch = [
        pltpu.VMEM((nch, _CL), jnp.int32),
        pltpu.VMEM((nch, _CL), jnp.int32),
        pltpu.VMEM((nch, _CL), jnp.int32),
        pltpu.VMEM((4, _CL, _CL), jnp.int32),
        pltpu.SemaphoreType.DMA,
        pltpu.SemaphoreType.DMA,
        pltpu.SemaphoreType.DMA,
        pltpu.SemaphoreType.DMA,
    ]

    def body(cell_hbm, adj_hbm, adjv_out, cell_v, rowid_v, adjv_v, bufs,
             *sems):
        cid = lax.axis_index("c")
        sid = lax.axis_index("s")
        wid = sid * _NC + cid
        pltpu.sync_copy(cell_hbm.at[wid], cell_v)
        iota = lax.iota(jnp.int32, 16)
        for j in range(nch):
            for g in range(8):
                sl = pl.ds(g * 16, 16)
                rowid_v[j, sl] = lax.shift_right_logical(
                    cell_v[j, sl], jnp.int32(7))

        def process(j, r):
            for g in range(8):
                sl = pl.ds(g * 16, 16)
                lane = lax.bitwise_and(cell_v[j, sl], jnp.int32(127))
                q16 = iota + g * 16
                adjv_v[j, sl] = plsc.load_gather(bufs.at[r], [q16, lane])

        assert nch % 4 == 0
        for r in range(4):
            pltpu.async_copy(adj_hbm.at[rowid_v.at[r]], bufs.at[r], sems[r])

        def loop_body(it, carry):
            j0 = it * 4
            for r in range(4):
                j = j0 + r
                pltpu.make_async_copy(
                    adj_hbm.at[rowid_v.at[0]], bufs.at[r], sems[r]).wait()
                process(j, r)

                @pl.when(j + 4 < nch)
                def _():
                    nxt = jnp.where(j + 4 < nch, j + 4, 0)
                    pltpu.async_copy(
                        adj_hbm.at[rowid_v.at[nxt]], bufs.at[r], sems[r])
            return carry

        lax.fori_loop(0, nch // 4, loop_body, 0)
        pltpu.sync_copy(adjv_v, adjv_out.at[wid])

    fn = pl.kernel(body, out_type=out_type, mesh=mesh,
                   scratch_types=scratch, interpret=interpret,
                   compiler_params=pltpu.CompilerParams(
                       needs_layout_passes=False))
    (adjv3,) = fn(cell3, adj2d)
    return adjv3


def _sc_corrections(rows3, cols3, b3, adjv3, s, d, m, h, fr,
                    interpret=False):
    n = s.shape[0]
    f2 = 16 * (-(-(fr + 1) // 16))
    dblk = fr // 16
    nw, nch, _ = rows3.shape
    sb = 8
    assert nw == _NW and nch % sb == 0 and h.shape[1] == 128
    nsb = nch // sb
    kz = -(-n // (_NS * _CL))
    n_pad = _NS * kz * _CL

    mesh = plsc.VectorSubcoreMesh(core_axis_name="c", subcore_axis_name="s")
    out_type = [jax.ShapeDtypeStruct((_NC, n_pad, f2), _F32)]
    scratch = [
        pltpu.VMEM((sb, _CL), jnp.int32),
        pltpu.VMEM((sb, _CL), jnp.int32),
        pltpu.VMEM((sb, _CL), _F32),
        pltpu.VMEM((sb, _CL), jnp.int32),
        pltpu.VMEM((n,), _F32),
        pltpu.VMEM((n,), _F32),
        pltpu.VMEM((n,), _F32),
        pltpu.VMEM((2, _CL, 128), _F32),
        pltpu.VMEM((_CL,), _F32),
        pltpu.VMEM((2, _CL, f2), _F32),
        pltpu.VMEM((_CL,), jnp.int32),
        pltpu.VMEM_SHARED((n, f2), _F32),
        pltpu.SemaphoreType.DMA,
        pltpu.SemaphoreType.DMA,
        pltpu.SemaphoreType.DMA,
        pltpu.SemaphoreType.DMA,
        pltpu.SemaphoreType.DMA,
    ]

    def body(rows_hbm, cols_hbm, b_hbm, adjv_hbm, s_hbm, d_hbm, m_hbm, h_hbm,
             out_hbm, rows_v, cols_v, b_v, adjv_v, s_v, d_v, m_v,
             hrs, delta_v, valss, idx_v, acc, semHA, semHB, semSA, semSB,
             semST):
        cid = lax.axis_index("c")
        sid = lax.axis_index("s")
        wid = sid * _NC + cid
        iota = lax.iota(jnp.int32, 16)

        pltpu.sync_copy(s_hbm, s_v)
        pltpu.sync_copy(d_hbm, d_v)
        pltpu.sync_copy(m_hbm, m_v)

        def build_idx(base):
            for g in range(8):
                idx_v[pl.ds(g * 16, 16)] = jnp.minimum(
                    iota + (base + g * 16), np.int32(n - 1))

        zero16 = jnp.zeros((16,), _F32)
        for r in range(_CL):
            for f0 in range(0, f2, 16):
                valss[0, r, pl.ds(f0, 16)] = zero16
        for k in range(kz):
            base = (sid * kz + k) * _CL
            build_idx(base)
            pltpu.sync_copy(valss.at[0], acc.at[idx_v])
        plsc.subcore_barrier()

        unit = jnp.where(iota == fr % 16, np.float32(1.0), np.float32(0.0))
        semH = (semHA, semHB)
        semS = (semSA, semSB)

        def process(j, p):
            hr = hrs.at[p]
            for g in range(8):
                sl = pl.ds(g * 16, 16)
                r16 = rows_v[j, sl]
                c16 = cols_v[j, sl]
                b16 = b_v[j, sl]
                a16 = adjv_v[j, sl]
                si = plsc.load_gather(s_v, [r16])
                dj = plsc.load_gather(d_v, [c16])
                mi = plsc.load_gather(m_v, [r16])
                a = si + dj
                p0 = jnp.exp(_leaky(a) - mi)
                p1 = jnp.exp(_leaky(a + b16) - mi)
                delta = jnp.where(a16 > 0, p1 - p0, np.float32(0.0))
                delta_v[sl] = delta
            for q in range(_CL):
                dq = plsc.load_gather(delta_v, [iota * 0 + q])
                for blk in range(f2 // 16):
                    hv = hr[q, pl.ds(blk * 16, 16)]
                    if blk == dblk:
                        hv = hv + unit
                    valss[p, q, pl.ds(blk * 16, 16)] = dq * hv
            pltpu.async_copy(valss.at[p], acc.at[rows_v.at[j]], semS[p],
                             add=True)

        def drain_scatter(p):
            pltpu.make_async_copy(
                valss.at[p], acc.at[rows_v.at[0]], semS[p]).wait()

        def fire_h(j, p):
            pltpu.async_copy(h_hbm.at[cols_v.at[j]], hrs.at[p], semH[p])

        def wait_h(p):
            pltpu.make_async_copy(
                h_hbm.at[cols_v.at[0]], hrs.at[p], semH[p]).wait()

        def sblock(u, carry):
            off = pl.multiple_of(u * sb, 8)
            usl = pl.ds(off, sb)
            cps = [pltpu.async_copy(rows_hbm.at[wid].at[usl], rows_v, semST),
                   pltpu.async_copy(cols_hbm.at[wid].at[usl], cols_v, semST),
                   pltpu.async_copy(b_hbm.at[wid].at[usl], b_v, semST),
                   pltpu.async_copy(adjv_hbm.at[wid].at[usl], adjv_v, semST)]
            for cp in cps:
                cp.wait()
            fire_h(0, 0)
            fire_h(1, 1)

            def loop_body(it, carry2):
                j = it * 2
                wait_h(0)

                @pl.when(it > 0)
                def _():
                    drain_scatter(0)
                process(j, 0)

                @pl.when(j + 2 < sb)
                def _():
                    fire_h(jnp.where(j + 2 < sb, j + 2, 0), 0)
                wait_h(1)

                @pl.when(it > 0)
                def _():
                    drain_scatter(1)
                process(j + 1, 1)

                @pl.when(j + 3 < sb)
                def _():
                    fire_h(jnp.where(j + 3 < sb, j + 3, 0), 1)
                return carry2

            lax.fori_loop(0, sb // 2, loop_body, 0)
            drain_scatter(0)
            drain_scatter(1)
            return carry

        lax.fori_loop(0, nsb, sblock, 0)

        plsc.subcore_barrier()
        for k in range(kz):
            c = sid * kz + k
            base = c * _CL
            build_idx(base)
            pltpu.sync_copy(acc.at[idx_v], valss.at[0])
            obase = pl.multiple_of(c * _CL, _CL)
            pltpu.sync_copy(valss.at[0], out_hbm.at[cid].at[pl.ds(obase, _CL)])

    fn = pl.kernel(body, out_type=out_type, mesh=mesh,
                   scratch_types=scratch, interpret=interpret,
                   compiler_params=pltpu.CompilerParams(
                       needs_layout_passes=False))
    (acc_out,) = fn(rows3, cols3, b3, adjv3, s, d, m, h)
    return acc_out[:, :n, :]


def _run(node_feats, edge_feats, edge_indices, adj, W0, a_src0, a_dst0, a_e0,
         W1, a_src1, a_dst1, a_e1, interpret=False):
    n = node_feats.shape[0]
    e = edge_feats.shape[0]
    hid = W0.shape[1]
    ncls = W1.shape[1]
    fr0 = hid
    fr1 = ncls
    fpad = 128
    bm = 80 if n % 80 == 0 else n

    rows = edge_indices[0].astype(jnp.int32)
    cols = edge_indices[1].astype(jnp.int32)
    ee0, ee1, cell = _edge_prep(edge_feats, a_e0, a_e1, rows, cols, n,
                                interpret=interpret)
    ept = -(-e // _NW)
    nch = -(-ept // _CL)
    nch += (-nch) % 16
    ep = _NW * nch * _CL

    def to3(x):
        return jnp.pad(x, (0, ep - e)).reshape(_NW, nch, _CL)

    rows3, cols3, cell3 = to3(rows), to3(cols), to3(cell)
    b03, b13 = to3(ee0), to3(ee1)
    adj2d = adj.reshape(-1, 128).astype(jnp.int32)
    adjv3 = _sc_adj_gather(cell3, adj2d, interpret=interpret)

    h0, hp0, s0, d0 = _prep_layer(node_feats, W0, a_src0, a_dst0, fpad,
                                  interpret=interpret)
    m0, den0, num0 = _dense_pass(s0, d0[:, 0], adj, h0, bm,
                                 interpret=interpret)
    acc0 = _sc_corrections(rows3, cols3, b03, adjv3, s0[:, 0], d0[:, 0],
                           m0[:, 0], hp0, fr0, interpret=interpret)
    accs0 = acc0[0] + acc0[1]
    dden0 = accs0[:, fr0]

    h1, hp1, s1, d1 = _combine_prep(num0, accs0, den0, dden0, W1, a_src1,
                                    a_dst1, fpad, interpret=interpret)
    m1, den1, num1 = _dense_pass(s1, d1[:, 0], adj, h1, bm,
                                 interpret=interpret)
    acc1 = _sc_corrections(rows3, cols3, b13, adjv3, s1[:, 0], d1[:, 0],
                           m1[:, 0], hp1, fr1, interpret=interpret)
    accs1 = acc1[0] + acc1[1]
    dden1 = accs1[:, fr1]

    return _final_combine(num1, accs1, den1, dden1, ncls,
                          interpret=interpret)


def kernel(node_feats, edge_feats, edge_indices, adj, W0, a_src0, a_dst0,
           a_e0, W1, a_src1, a_dst1, a_e1):
    return _run(node_feats, edge_feats, edge_indices, adj, W0, a_src0,
                a_dst0, a_e0, W1, a_src1, a_dst1, a_e1)


# speedup vs baseline: 1.1853x; 1.0262x over previous
import functools

import jax
import jax.numpy as jnp
from jax import lax
from jax.experimental import pallas as pl
from jax.experimental.pallas import tpu as pltpu
from jax.experimental.pallas import tpu_sc as plsc

_NC = 2
_NS = 16
_NW = _NC * _NS
_CL = 128

import numpy as np

_NEG = np.float32(-9e15)
_F32 = jnp.float32
_HI = jax.lax.Precision.HIGHEST


def _dot(a, b):
    return jax.lax.dot_general(a, b, (((1,), (0,)), ((), ())),
                               preferred_element_type=jnp.float32,
                               precision=_HI)


def _leaky(x):
    return jnp.where(x >= 0, x, jnp.float32(0.2) * x)


def _prep_layer(x, W, a_src, a_dst, f_pad, interpret=False):
    n, _ = x.shape
    f = W.shape[1]
    bm = 1000 if n % 1000 == 0 else n

    def body(x_ref, w_ref, as_ref, ad_ref, h_ref, hp_ref, s_ref, d_ref):
        h = _dot(x_ref[...], w_ref[...])
        s_ref[...] = _dot(h, as_ref[...])
        d_ref[...] = _dot(h, ad_ref[...])
        h_ref[...] = h
        hp_ref[...] = jnp.concatenate(
            [h, jnp.zeros((h.shape[0], f_pad - f), _F32)], axis=1)

    h, hp, s, d = pl.pallas_call(
        body,
        grid=(n // bm,),
        in_specs=[
            pl.BlockSpec((bm, x.shape[1]), lambda i: (i, 0)),
            pl.BlockSpec((W.shape[0], f), lambda i: (0, 0)),
            pl.BlockSpec((f, 1), lambda i: (0, 0)),
            pl.BlockSpec((f, 1), lambda i: (0, 0)),
        ],
        out_specs=[
            pl.BlockSpec((bm, f), lambda i: (i, 0)),
            pl.BlockSpec((bm, f_pad), lambda i: (i, 0)),
            pl.BlockSpec((bm, 1), lambda i: (i, 0)),
            pl.BlockSpec((bm, 1), lambda i: (i, 0)),
        ],
        out_shape=[
            jax.ShapeDtypeStruct((n, f), _F32),
            jax.ShapeDtypeStruct((n, f_pad), _F32),
            jax.ShapeDtypeStruct((n, 1), _F32),
            jax.ShapeDtypeStruct((n, 1), _F32),
        ],
        interpret=interpret,
    )(x, W, a_src.reshape(-1, 1), a_dst.reshape(-1, 1))
    return h, hp, s, d


def _edge_prep(edge_feats, a_e0, a_e1, rows, cols, n, interpret=False):
    e, k = edge_feats.shape
    be = 1000 if e % 1000 == 0 else e

    def body(ef_ref, a0_ref, a1_ref, r_ref, c_ref, o0_ref, o1_ref, cell_ref):
        o0_ref[...] = _dot(ef_ref[...], a0_ref[...])
        o1_ref[...] = _dot(ef_ref[...], a1_ref[...])
        cell_ref[...] = r_ref[...] * np.int32(n) + c_ref[...]

    ee0, ee1, cell = pl.pallas_call(
        body,
        grid=(e // be,),
        in_specs=[
            pl.BlockSpec((be, k), lambda i: (i, 0)),
            pl.BlockSpec((k, 1), lambda i: (0, 0)),
            pl.BlockSpec((k, 1), lambda i: (0, 0)),
            pl.BlockSpec((be, 1), lambda i: (i, 0)),
            pl.BlockSpec((be, 1), lambda i: (i, 0)),
        ],
        out_specs=[
            pl.BlockSpec((be, 1), lambda i: (i, 0)),
            pl.BlockSpec((be, 1), lambda i: (i, 0)),
            pl.BlockSpec((be, 1), lambda i: (i, 0)),
        ],
        out_shape=[
            jax.ShapeDtypeStruct((e, 1), _F32),
            jax.ShapeDtypeStruct((e, 1), _F32),
            jax.ShapeDtypeStruct((e, 1), jnp.int32),
        ],
        interpret=interpret,
    )(edge_feats, a_e0.reshape(-1, 1), a_e1.reshape(-1, 1),
      rows.reshape(-1, 1), cols.reshape(-1, 1))
    return ee0[:, 0], ee1[:, 0], cell[:, 0]


def _dense_pass(s, d, adj, h, bm, interpret=False):
    n = adj.shape[0]
    f = h.shape[1]

    def body(s_ref, d_ref, adj_ref, h_ref, m_ref, den_ref, num_ref):
        a = s_ref[...] + d_ref[...]
        e0 = _leaky(a)
        masked = jnp.where(adj_ref[...] > 0, e0, _NEG)
        m = jnp.max(masked, axis=1, keepdims=True)
        p = jnp.exp(masked - m)
        m_ref[...] = m
        den_ref[...] = jnp.sum(p, axis=1, keepdims=True)
        num_ref[...] = _dot(p, h_ref[...])

    m, den, num = pl.pallas_call(
        body,
        grid=(n // bm,),
        in_specs=[
            pl.BlockSpec((bm, 1), lambda i: (i, 0)),
            pl.BlockSpec((1, n), lambda i: (0, 0)),
            pl.BlockSpec((bm, n), lambda i: (i, 0)),
            pl.BlockSpec((n, f), lambda i: (0, 0)),
        ],
        out_specs=[
            pl.BlockSpec((bm, 1), lambda i: (i, 0)),
            pl.BlockSpec((bm, 1), lambda i: (i, 0)),
            pl.BlockSpec((bm, f), lambda i: (i, 0)),
        ],
        out_shape=[
            jax.ShapeDtypeStruct((n, 1), _F32),
            jax.ShapeDtypeStruct((n, 1), _F32),
            jax.ShapeDtypeStruct((n, f), _F32),
        ],
        interpret=interpret,
    )(s, d.reshape(1, -1), adj, h)
    return m, den, num


def _final_combine(num, dnum, den, dden, f_out, interpret=False):
    n = num.shape[0]
    bm = 1000 if n % 1000 == 0 else n

    def body(num_ref, dnum_ref, den_ref, dden_ref, o_ref):
        x = ((num_ref[...][:, :f_out] + dnum_ref[...][:, :f_out])
             / (den_ref[...] + dden_ref[...]))
        o_ref[...] = jnp.where(x > 0, x, jnp.exp(x) - np.float32(1.0))

    return pl.pallas_call(
        body,
        grid=(n // bm,),
        in_specs=[
            pl.BlockSpec((bm, num.shape[1]), lambda i: (i, 0)),
            pl.BlockSpec((bm, dnum.shape[1]), lambda i: (i, 0)),
            pl.BlockSpec((bm, 1), lambda i: (i, 0)),
            pl.BlockSpec((bm, 1), lambda i: (i, 0)),
        ],
        out_specs=pl.BlockSpec((bm, f_out), lambda i: (i, 0)),
        out_shape=jax.ShapeDtypeStruct((n, f_out), _F32),
        interpret=interpret,
    )(num, dnum, den, dden.reshape(-1, 1) if dden.ndim == 1 else dden)


def _combine_prep(num, dnum, den, dden, W, a_src, a_dst, f_pad,
                  interpret=False):
    n = num.shape[0]
    f_in = W.shape[0]
    f = W.shape[1]
    bm = 1000 if n % 1000 == 0 else n

    def body(num_ref, dnum_ref, den_ref, dden_ref, w_ref, as_ref, ad_ref,
             h_ref, hp_ref, s_ref, d_ref):
        x = ((num_ref[...][:, :f_in] + dnum_ref[...][:, :f_in])
             / (den_ref[...] + dden_ref[...]))
        h = _dot(x, w_ref[...])
        s_ref[...] = _dot(h, as_ref[...])
        d_ref[...] = _dot(h, ad_ref[...])
        h_ref[...] = h
        hp_ref[...] = jnp.concatenate(
            [h, jnp.zeros((h.shape[0], f_pad - f), _F32)], axis=1)

    h, hp, s, d = pl.pallas_call(
        body,
        grid=(n // bm,),
        in_specs=[
            pl.BlockSpec((bm, num.shape[1]), lambda i: (i, 0)),
            pl.BlockSpec((bm, dnum.shape[1]), lambda i: (i, 0)),
            pl.BlockSpec((bm, 1), lambda i: (i, 0)),
            pl.BlockSpec((bm, 1), lambda i: (i, 0)),
            pl.BlockSpec((f_in, f), lambda i: (0, 0)),
            pl.BlockSpec((f, 1), lambda i: (0, 0)),
            pl.BlockSpec((f, 1), lambda i: (0, 0)),
        ],
        out_specs=[
            pl.BlockSpec((bm, f), lambda i: (i, 0)),
            pl.BlockSpec((bm, f_pad), lambda i: (i, 0)),
            pl.BlockSpec((bm, 1), lambda i: (i, 0)),
            pl.BlockSpec((bm, 1), lambda i: (i, 0)),
        ],
        out_shape=[
            jax.ShapeDtypeStruct((n, f), _F32),
            jax.ShapeDtypeStruct((n, f_pad), _F32),
            jax.ShapeDtypeStruct((n, 1), _F32),
            jax.ShapeDtypeStruct((n, 1), _F32),
        ],
        interpret=interpret,
    )(num, dnum, den,
      dden.reshape(-1, 1) if dden.ndim == 1 else dden,
      W, a_src.reshape(-1, 1), a_dst.reshape(-1, 1))
    return h, hp, s, d


def _sc_adj_gather(cell3, adj2d, interpret=False):
    nw, nch, _ = cell3.shape
    mesh = plsc.VectorSubcoreMesh(core_axis_name="c", subcore_axis_name="s")
    out_type = [jax.ShapeDtypeStruct((nw, nch, _CL), jnp.int32)]
    scratch = [
        pltpu.VMEM((nch, _CL), jnp.int32),
        pltpu.VMEM((nch, _CL), jnp.int32),
        pltpu.VMEM((nch, _CL), jnp.int32),
        pltpu.VMEM((4, _CL, _CL), jnp.int32),
        pltpu.SemaphoreType.DMA,
        pltpu.SemaphoreType.DMA,
        pltpu.SemaphoreType.DMA,
        pltpu.SemaphoreType.DMA,
    ]

    def body(cell_hbm, adj_hbm, adjv_out, cell_v, rowid_v, adjv_v, bufs,
             *sems):
        cid = lax.axis_index("c")
        sid = lax.axis_index("s")
        wid = sid * _NC + cid
        pltpu.sync_copy(cell_hbm.at[wid], cell_v)
        iota = lax.iota(jnp.int32, 16)
        for j in range(nch):
            for g in range(8):
                sl = pl.ds(g * 16, 16)
                rowid_v[j, sl] = lax.shift_right_logical(
                    cell_v[j, sl], jnp.int32(7))

        def process(j, r):
            for g in range(8):
                sl = pl.ds(g * 16, 16)
                lane = lax.bitwise_and(cell_v[j, sl], jnp.int32(127))
                q16 = iota + g * 16
                adjv_v[j, sl] = plsc.load_gather(bufs.at[r], [q16, lane])

        assert nch % 4 == 0
        for r in range(4):
            pltpu.async_copy(adj_hbm.at[rowid_v.at[r]], bufs.at[r], sems[r])

        def loop_body(it, carry):
            j0 = it * 4
            for r in range(4):
                j = j0 + r
                pltpu.make_async_copy(
                    adj_hbm.at[rowid_v.at[0]], bufs.at[r], sems[r]).wait()
                process(j, r)

                @pl.when(j + 4 < nch)
                def _():
                    nxt = jnp.where(j + 4 < nch, j + 4, 0)
                    pltpu.async_copy(
                        adj_hbm.at[rowid_v.at[nxt]], bufs.at[r], sems[r])
            return carry

        lax.fori_loop(0, nch // 4, loop_body, 0)
        pltpu.sync_copy(adjv_v, adjv_out.at[wid])

    fn = pl.kernel(body, out_type=out_type, mesh=mesh,
                   scratch_types=scratch, interpret=interpret,
                   compiler_params=pltpu.CompilerParams(
                       needs_layout_passes=False))
    (adjv3,) = fn(cell3, adj2d)
    return adjv3


def _sc_corrections(rows3, cols3, b3, adjv3, s, d, m, h, fr,
                    interpret=False):
    n = s.shape[0]
    f2 = 16 * (-(-(fr + 1) // 16))
    dblk = fr // 16
    nw, nch, _ = rows3.shape
    sb = 8
    assert nw == _NW and nch % sb == 0 and h.shape[1] == 128
    nsb = nch // sb
    kz = -(-n // (_NS * _CL))
    n_pad = _NS * kz * _CL

    mesh = plsc.VectorSubcoreMesh(core_axis_name="c", subcore_axis_name="s")
    out_type = [jax.ShapeDtypeStruct((_NC, n_pad, f2), _F32)]
    scratch = [
        pltpu.VMEM((sb, _CL), jnp.int32),
        pltpu.VMEM((sb, _CL), jnp.int32),
        pltpu.VMEM((sb, _CL), _F32),
        pltpu.VMEM((sb, _CL), jnp.int32),
        pltpu.VMEM((n,), _F32),
        pltpu.VMEM((n,), _F32),
        pltpu.VMEM((n,), _F32),
        pltpu.VMEM((2, _CL, 128), _F32),
        pltpu.VMEM((_CL,), _F32),
        pltpu.VMEM((2, _CL, f2), _F32),
        pltpu.VMEM((_CL,), jnp.int32),
        pltpu.VMEM_SHARED((n, f2), _F32),
        pltpu.SemaphoreType.DMA,
        pltpu.SemaphoreType.DMA,
        pltpu.SemaphoreType.DMA,
        pltpu.SemaphoreType.DMA,
        pltpu.SemaphoreType.DMA,
    ]

    def body(rows_hbm, cols_hbm, b_hbm, adjv_hbm, s_hbm, d_hbm, m_hbm, h_hbm,
             out_hbm, rows_v, cols_v, b_v, adjv_v, s_v, d_v, m_v,
             hrs, delta_v, valss, idx_v, acc, semHA, semHB, semSA, semSB,
             semST):
        cid = lax.axis_index("c")
        sid = lax.axis_index("s")
        wid = sid * _NC + cid
        iota = lax.iota(jnp.int32, 16)

        pltpu.sync_copy(s_hbm, s_v)
        pltpu.sync_copy(d_hbm, d_v)
        pltpu.sync_copy(m_hbm, m_v)

        def build_idx(base):
            for g in range(8):
                idx_v[pl.ds(g * 16, 16)] = jnp.minimum(
                    iota + (base + g * 16), np.int32(n - 1))

        zero16 = jnp.zeros((16,), _F32)
        for r in range(_CL):
            for f0 in range(0, f2, 16):
                valss[0, r, pl.ds(f0, 16)] = zero16
        for k in range(kz):
            base = (sid * kz + k) * _CL
            build_idx(base)
            pltpu.sync_copy(valss.at[0], acc.at[idx_v])
        plsc.subcore_barrier()

        unit = jnp.where(iota == fr % 16, np.float32(1.0), np.float32(0.0))
        semH = (semHA, semHB)
        semS = (semSA, semSB)

        def process(j, p):
            hr = hrs.at[p]
            for g in range(8):
                sl = pl.ds(g * 16, 16)
                r16 = rows_v[j, sl]
                c16 = cols_v[j, sl]
                b16 = b_v[j, sl]
                a16 = adjv_v[j, sl]
                si = plsc.load_gather(s_v, [r16])
                dj = plsc.load_gather(d_v, [c16])
                mi = plsc.load_gather(m_v, [r16])
                a = si + dj
                p0 = jnp.exp(_leaky(a) - mi)
                p1 = jnp.exp(_leaky(a + b16) - mi)
                delta = jnp.where(a16 > 0, p1 - p0, np.float32(0.0))
                delta_v[sl] = delta
            for q in range(_CL):
                dq = plsc.load_gather(delta_v, [iota * 0 + q])
                for blk in range(f2 // 16):
                    hv = hr[q, pl.ds(blk * 16, 16)]
                    if blk == dblk:
                        hv = hv + unit
                    valss[p, q, pl.ds(blk * 16, 16)] = dq * hv
            pltpu.async_copy(valss.at[p], acc.at[rows_v.at[j]], semS[p],
                             add=True)

        def drain_scatter(p):
            pltpu.make_async_copy(
                valss.at[p], acc.at[rows_v.at[0]], semS[p]).wait()

        def fire_h(j, p):
            pltpu.async_copy(h_hbm.at[cols_v.at[j]], hrs.at[p], semH[p])

        def wait_h(p):
            pltpu.make_async_copy(
                h_hbm.at[cols_v.at[0]], hrs.at[p], semH[p]).wait()

        def sblock(u, carry):
            off = pl.multiple_of(u * sb, 8)
            usl = pl.ds(off, sb)
            cps = [pltpu.async_copy(rows_hbm.at[wid].at[usl], rows_v, semST),
                   pltpu.async_copy(cols_hbm.at[wid].at[usl], cols_v, semST),
                   pltpu.async_copy(b_hbm.at[wid].at[usl], b_v, semST),
                   pltpu.async_copy(adjv_hbm.at[wid].at[usl], adjv_v, semST)]
            for cp in cps:
                cp.wait()
            fire_h(0, 0)
            fire_h(1, 1)

            def loop_body(it, carry2):
                j = it * 2
                wait_h(0)

                @pl.when(it > 0)
                def _():
                    drain_scatter(0)
                process(j, 0)

                @pl.when(j + 2 < sb)
                def _():
                    fire_h(jnp.where(j + 2 < sb, j + 2, 0), 0)
                wait_h(1)

                @pl.when(it > 0)
                def _():
                    drain_scatter(1)
                process(j + 1, 1)

                @pl.when(j + 3 < sb)
                def _():
                    fire_h(jnp.where(j + 3 < sb, j + 3, 0), 1)
                return carry2

            lax.fori_loop(0, sb // 2, loop_body, 0)
            drain_scatter(0)
            drain_scatter(1)
            return carry

        lax.fori_loop(0, nsb, sblock, 0)

        plsc.subcore_barrier()
        for k in range(kz):
            c = sid * kz + k
            base = c * _CL
            build_idx(base)
            pltpu.sync_copy(acc.at[idx_v], valss.at[0])
            obase = pl.multiple_of(c * _CL, _CL)
            pltpu.sync_copy(valss.at[0], out_hbm.at[cid].at[pl.ds(obase, _CL)])

    fn = pl.kernel(body, out_type=out_type, mesh=mesh,
                   scratch_types=scratch, interpret=interpret,
                   compiler_params=pltpu.CompilerParams(
                       needs_layout_passes=False))
    (acc_out,) = fn(rows3, cols3, b3, adjv3, s, d, m, h)
    return acc_out[:, :n, :]


def _run(node_feats, edge_feats, edge_indices, adj, W0, a_src0, a_dst0, a_e0,
         W1, a_src1, a_dst1, a_e1, interpret=False):
    n = node_feats.shape[0]
    e = edge_feats.shape[0]
    hid = W0.shape[1]
    ncls = W1.shape[1]
    fr0 = hid
    fr1 = ncls
    fpad = 128
    bm = 200 if n % 200 == 0 else n

    rows = edge_indices[0].astype(jnp.int32)
    cols = edge_indices[1].astype(jnp.int32)
    ee0, ee1, cell = _edge_prep(edge_feats, a_e0, a_e1, rows, cols, n,
                                interpret=interpret)
    ept = -(-e // _NW)
    nch = -(-ept // _CL)
    nch += (-nch) % 16
    ep = _NW * nch * _CL

    def to3(x):
        return jnp.pad(x, (0, ep - e)).reshape(_NW, nch, _CL)

    rows3, cols3, cell3 = to3(rows), to3(cols), to3(cell)
    b03, b13 = to3(ee0), to3(ee1)
    adj2d = adj.reshape(-1, 128).astype(jnp.int32)
    adjv3 = _sc_adj_gather(cell3, adj2d, interpret=interpret)

    h0, hp0, s0, d0 = _prep_layer(node_feats, W0, a_src0, a_dst0, fpad,
                                  interpret=interpret)
    m0, den0, num0 = _dense_pass(s0, d0[:, 0], adj, h0, bm,
                                 interpret=interpret)
    acc0 = _sc_corrections(rows3, cols3, b03, adjv3, s0[:, 0], d0[:, 0],
                           m0[:, 0], hp0, fr0, interpret=interpret)
    accs0 = acc0[0] + acc0[1]
    dden0 = accs0[:, fr0]

    h1, hp1, s1, d1 = _combine_prep(num0, accs0, den0, dden0, W1, a_src1,
                                    a_dst1, fpad, interpret=interpret)
    m1, den1, num1 = _dense_pass(s1, d1[:, 0], adj, h1, bm,
                                 interpret=interpret)
    acc1 = _sc_corrections(rows3, cols3, b13, adjv3, s1[:, 0], d1[:, 0],
                           m1[:, 0], hp1, fr1, interpret=interpret)
    accs1 = acc1[0] + acc1[1]
    dden1 = accs1[:, fr1]

    return _final_combine(num1, accs1, den1, dden1, ncls,
                          interpret=interpret)


def kernel(node_feats, edge_feats, edge_indices, adj, W0, a_src0, a_dst0,
           a_e0, W1, a_src1, a_dst1, a_e1):
    return _run(node_feats, edge_feats, edge_indices, adj, W0, a_src0,
                a_dst0, a_e0, W1, a_src1, a_dst1, a_e1)


# speedup vs baseline: 1.2167x; 1.0264x over previous
import functools

import jax
import jax.numpy as jnp
from jax import lax
from jax.experimental import pallas as pl
from jax.experimental.pallas import tpu as pltpu
from jax.experimental.pallas import tpu_sc as plsc

_NC = 2
_NS = 16
_NW = _NC * _NS
_CL = 128

import numpy as np

_NEG = np.float32(-9e15)
_F32 = jnp.float32
_HI = jax.lax.Precision.HIGHEST


def _dot(a, b):
    return jax.lax.dot_general(a, b, (((1,), (0,)), ((), ())),
                               preferred_element_type=jnp.float32,
                               precision=_HI)


def _leaky(x):
    return jnp.where(x >= 0, x, jnp.float32(0.2) * x)


def _prep_layer(x, W, a_src, a_dst, f_pad, interpret=False):
    n, _ = x.shape
    f = W.shape[1]
    bm = 1000 if n % 1000 == 0 else n

    def body(x_ref, w_ref, as_ref, ad_ref, h_ref, hp_ref, s_ref, d_ref):
        h = _dot(x_ref[...], w_ref[...])
        s_ref[...] = _dot(h, as_ref[...])
        d_ref[...] = _dot(h, ad_ref[...])
        h_ref[...] = h
        hp_ref[...] = jnp.concatenate(
            [h, jnp.zeros((h.shape[0], f_pad - f), _F32)], axis=1)

    h, hp, s, d = pl.pallas_call(
        body,
        grid=(n // bm,),
        in_specs=[
            pl.BlockSpec((bm, x.shape[1]), lambda i: (i, 0)),
            pl.BlockSpec((W.shape[0], f), lambda i: (0, 0)),
            pl.BlockSpec((f, 1), lambda i: (0, 0)),
            pl.BlockSpec((f, 1), lambda i: (0, 0)),
        ],
        out_specs=[
            pl.BlockSpec((bm, f), lambda i: (i, 0)),
            pl.BlockSpec((bm, f_pad), lambda i: (i, 0)),
            pl.BlockSpec((bm, 1), lambda i: (i, 0)),
            pl.BlockSpec((bm, 1), lambda i: (i, 0)),
        ],
        out_shape=[
            jax.ShapeDtypeStruct((n, f), _F32),
            jax.ShapeDtypeStruct((n, f_pad), _F32),
            jax.ShapeDtypeStruct((n, 1), _F32),
            jax.ShapeDtypeStruct((n, 1), _F32),
        ],
        interpret=interpret,
    )(x, W, a_src.reshape(-1, 1), a_dst.reshape(-1, 1))
    return h, hp, s, d


def _edge_prep(edge_feats, a_e0, a_e1, rows, cols, n, interpret=False):
    e, k = edge_feats.shape
    be = 1000 if e % 1000 == 0 else e

    def body(ef_ref, a0_ref, a1_ref, r_ref, c_ref, o0_ref, o1_ref, cell_ref):
        o0_ref[...] = _dot(ef_ref[...], a0_ref[...])
        o1_ref[...] = _dot(ef_ref[...], a1_ref[...])
        cell_ref[...] = r_ref[...] * np.int32(n) + c_ref[...]

    ee0, ee1, cell = pl.pallas_call(
        body,
        grid=(e // be,),
        in_specs=[
            pl.BlockSpec((be, k), lambda i: (i, 0)),
            pl.BlockSpec((k, 1), lambda i: (0, 0)),
            pl.BlockSpec((k, 1), lambda i: (0, 0)),
            pl.BlockSpec((be, 1), lambda i: (i, 0)),
            pl.BlockSpec((be, 1), lambda i: (i, 0)),
        ],
        out_specs=[
            pl.BlockSpec((be, 1), lambda i: (i, 0)),
            pl.BlockSpec((be, 1), lambda i: (i, 0)),
            pl.BlockSpec((be, 1), lambda i: (i, 0)),
        ],
        out_shape=[
            jax.ShapeDtypeStruct((e, 1), _F32),
            jax.ShapeDtypeStruct((e, 1), _F32),
            jax.ShapeDtypeStruct((e, 1), jnp.int32),
        ],
        interpret=interpret,
    )(edge_feats, a_e0.reshape(-1, 1), a_e1.reshape(-1, 1),
      rows.reshape(-1, 1), cols.reshape(-1, 1))
    return ee0[:, 0], ee1[:, 0], cell[:, 0]


def _dense_pass(s, d, adj, h, bm, interpret=False):
    n = adj.shape[0]
    f = h.shape[1]

    def body(s_ref, d_ref, adj_ref, h_ref, m_ref, den_ref, num_ref):
        a = s_ref[...] + d_ref[...]
        e0 = _leaky(a)
        masked = jnp.where(adj_ref[...] > 0, e0, _NEG)
        m = jnp.max(masked, axis=1, keepdims=True)
        p = jnp.exp(masked - m)
        m_ref[...] = m
        den_ref[...] = jnp.sum(p, axis=1, keepdims=True)
        num_ref[...] = _dot(p, h_ref[...])

    m, den, num = pl.pallas_call(
        body,
        grid=(n // bm,),
        in_specs=[
            pl.BlockSpec((bm, 1), lambda i: (i, 0)),
            pl.BlockSpec((1, n), lambda i: (0, 0)),
            pl.BlockSpec((bm, n), lambda i: (i, 0)),
            pl.BlockSpec((n, f), lambda i: (0, 0)),
        ],
        out_specs=[
            pl.BlockSpec((bm, 1), lambda i: (i, 0)),
            pl.BlockSpec((bm, 1), lambda i: (i, 0)),
            pl.BlockSpec((bm, f), lambda i: (i, 0)),
        ],
        out_shape=[
            jax.ShapeDtypeStruct((n, 1), _F32),
            jax.ShapeDtypeStruct((n, 1), _F32),
            jax.ShapeDtypeStruct((n, f), _F32),
        ],
        interpret=interpret,
    )(s, d.reshape(1, -1), adj, h)
    return m, den, num


def _final_combine(num, dnum, den, dden, f_out, interpret=False):
    n = num.shape[0]
    bm = 1000 if n % 1000 == 0 else n

    def body(num_ref, dnum_ref, den_ref, dden_ref, o_ref):
        x = ((num_ref[...][:, :f_out] + dnum_ref[...][:, :f_out])
             / (den_ref[...] + dden_ref[...]))
        o_ref[...] = jnp.where(x > 0, x, jnp.exp(x) - np.float32(1.0))

    return pl.pallas_call(
        body,
        grid=(n // bm,),
        in_specs=[
            pl.BlockSpec((bm, num.shape[1]), lambda i: (i, 0)),
            pl.BlockSpec((bm, dnum.shape[1]), lambda i: (i, 0)),
            pl.BlockSpec((bm, 1), lambda i: (i, 0)),
            pl.BlockSpec((bm, 1), lambda i: (i, 0)),
        ],
        out_specs=pl.BlockSpec((bm, f_out), lambda i: (i, 0)),
        out_shape=jax.ShapeDtypeStruct((n, f_out), _F32),
        interpret=interpret,
    )(num, dnum, den, dden.reshape(-1, 1) if dden.ndim == 1 else dden)


def _combine_prep(num, dnum, den, dden, W, a_src, a_dst, f_pad,
                  interpret=False):
    n = num.shape[0]
    f_in = W.shape[0]
    f = W.shape[1]
    bm = 1000 if n % 1000 == 0 else n

    def body(num_ref, dnum_ref, den_ref, dden_ref, w_ref, as_ref, ad_ref,
             h_ref, hp_ref, s_ref, d_ref):
        x = ((num_ref[...][:, :f_in] + dnum_ref[...][:, :f_in])
             / (den_ref[...] + dden_ref[...]))
        h = _dot(x, w_ref[...])
        s_ref[...] = _dot(h, as_ref[...])
        d_ref[...] = _dot(h, ad_ref[...])
        h_ref[...] = h
        hp_ref[...] = jnp.concatenate(
            [h, jnp.zeros((h.shape[0], f_pad - f), _F32)], axis=1)

    h, hp, s, d = pl.pallas_call(
        body,
        grid=(n // bm,),
        in_specs=[
            pl.BlockSpec((bm, num.shape[1]), lambda i: (i, 0)),
            pl.BlockSpec((bm, dnum.shape[1]), lambda i: (i, 0)),
            pl.BlockSpec((bm, 1), lambda i: (i, 0)),
            pl.BlockSpec((bm, 1), lambda i: (i, 0)),
            pl.BlockSpec((f_in, f), lambda i: (0, 0)),
            pl.BlockSpec((f, 1), lambda i: (0, 0)),
            pl.BlockSpec((f, 1), lambda i: (0, 0)),
        ],
        out_specs=[
            pl.BlockSpec((bm, f), lambda i: (i, 0)),
            pl.BlockSpec((bm, f_pad), lambda i: (i, 0)),
            pl.BlockSpec((bm, 1), lambda i: (i, 0)),
            pl.BlockSpec((bm, 1), lambda i: (i, 0)),
        ],
        out_shape=[
            jax.ShapeDtypeStruct((n, f), _F32),
            jax.ShapeDtypeStruct((n, f_pad), _F32),
            jax.ShapeDtypeStruct((n, 1), _F32),
            jax.ShapeDtypeStruct((n, 1), _F32),
        ],
        interpret=interpret,
    )(num, dnum, den,
      dden.reshape(-1, 1) if dden.ndim == 1 else dden,
      W, a_src.reshape(-1, 1), a_dst.reshape(-1, 1))
    return h, hp, s, d


def _sc_adj_gather(cell3, adjflat, interpret=False):
    nw, nch, _ = cell3.shape
    mesh = plsc.VectorSubcoreMesh(core_axis_name="c", subcore_axis_name="s")
    out_type = [jax.ShapeDtypeStruct((nw, nch, _CL), jnp.int32)]
    scratch = [
        pltpu.VMEM((nch, _CL), jnp.int32),
        pltpu.VMEM((nch, _CL), jnp.int32),
        pltpu.SemaphoreType.DMA,
    ]

    def body(cell_hbm, adj_hbm, adjv_out, cell_v, adjv_v, sem):
        cid = lax.axis_index("c")
        sid = lax.axis_index("s")
        wid = sid * _NC + cid
        pltpu.sync_copy(cell_hbm.at[wid], cell_v)

        assert nch % 16 == 0
        for w in range(nch // 16):
            cps = [pltpu.async_copy(adj_hbm.at[cell_v.at[w * 16 + i]],
                                    adjv_v.at[w * 16 + i], sem)
                   for i in range(16)]
            for cp in cps:
                cp.wait()
        pltpu.sync_copy(adjv_v, adjv_out.at[wid])

    fn = pl.kernel(body, out_type=out_type, mesh=mesh,
                   scratch_types=scratch, interpret=interpret,
                   compiler_params=pltpu.CompilerParams(
                       needs_layout_passes=False))
    (adjv3,) = fn(cell3, adjflat)
    return adjv3


def _sc_corrections(rows3, cols3, b3, adjv3, s, d, m, h, fr,
                    interpret=False):
    n = s.shape[0]
    f2 = 16 * (-(-(fr + 1) // 16))
    dblk = fr // 16
    nw, nch, _ = rows3.shape
    sb = 8
    assert nw == _NW and nch % sb == 0 and h.shape[1] == 128
    nsb = nch // sb
    kz = -(-n // (_NS * _CL))
    n_pad = _NS * kz * _CL

    mesh = plsc.VectorSubcoreMesh(core_axis_name="c", subcore_axis_name="s")
    out_type = [jax.ShapeDtypeStruct((_NC, n_pad, f2), _F32)]
    scratch = [
        pltpu.VMEM((sb, _CL), jnp.int32),
        pltpu.VMEM((sb, _CL), jnp.int32),
        pltpu.VMEM((sb, _CL), _F32),
        pltpu.VMEM((sb, _CL), jnp.int32),
        pltpu.VMEM((n,), _F32),
        pltpu.VMEM((n,), _F32),
        pltpu.VMEM((n,), _F32),
        pltpu.VMEM((2, _CL, 128), _F32),
        pltpu.VMEM((_CL,), _F32),
        pltpu.VMEM((2, _CL, f2), _F32),
        pltpu.VMEM((_CL,), jnp.int32),
        pltpu.VMEM_SHARED((n, f2), _F32),
        pltpu.SemaphoreType.DMA,
        pltpu.SemaphoreType.DMA,
        pltpu.SemaphoreType.DMA,
        pltpu.SemaphoreType.DMA,
        pltpu.SemaphoreType.DMA,
    ]

    def body(rows_hbm, cols_hbm, b_hbm, adjv_hbm, s_hbm, d_hbm, m_hbm, h_hbm,
             out_hbm, rows_v, cols_v, b_v, adjv_v, s_v, d_v, m_v,
             hrs, delta_v, valss, idx_v, acc, semHA, semHB, semSA, semSB,
             semST):
        cid = lax.axis_index("c")
        sid = lax.axis_index("s")
        wid = sid * _NC + cid
        iota = lax.iota(jnp.int32, 16)

        pltpu.sync_copy(s_hbm, s_v)
        pltpu.sync_copy(d_hbm, d_v)
        pltpu.sync_copy(m_hbm, m_v)

        def build_idx(base):
            for g in range(8):
                idx_v[pl.ds(g * 16, 16)] = jnp.minimum(
                    iota + (base + g * 16), np.int32(n - 1))

        zero16 = jnp.zeros((16,), _F32)
        for r in range(_CL):
            for f0 in range(0, f2, 16):
                valss[0, r, pl.ds(f0, 16)] = zero16
        for k in range(kz):
            base = (sid * kz + k) * _CL
            build_idx(base)
            pltpu.sync_copy(valss.at[0], acc.at[idx_v])
        plsc.subcore_barrier()

        unit = jnp.where(iota == fr % 16, np.float32(1.0), np.float32(0.0))
        semH = (semHA, semHB)
        semS = (semSA, semSB)

        def process(j, p):
            hr = hrs.at[p]
            for g in range(8):
                sl = pl.ds(g * 16, 16)
                r16 = rows_v[j, sl]
                c16 = cols_v[j, sl]
                b16 = b_v[j, sl]
                a16 = adjv_v[j, sl]
                si = plsc.load_gather(s_v, [r16])
                dj = plsc.load_gather(d_v, [c16])
                mi = plsc.load_gather(m_v, [r16])
                a = si + dj
                p0 = jnp.exp(_leaky(a) - mi)
                p1 = jnp.exp(_leaky(a + b16) - mi)
                delta = jnp.where(a16 > 0, p1 - p0, np.float32(0.0))
                delta_v[sl] = delta
            for q in range(_CL):
                dq = plsc.load_gather(delta_v, [iota * 0 + q])
                for blk in range(f2 // 16):
                    hv = hr[q, pl.ds(blk * 16, 16)]
                    if blk == dblk:
                        hv = hv + unit
                    valss[p, q, pl.ds(blk * 16, 16)] = dq * hv
            pltpu.async_copy(valss.at[p], acc.at[rows_v.at[j]], semS[p],
                             add=True)

        def drain_scatter(p):
            pltpu.make_async_copy(
                valss.at[p], acc.at[rows_v.at[0]], semS[p]).wait()

        def fire_h(j, p):
            pltpu.async_copy(h_hbm.at[cols_v.at[j]], hrs.at[p], semH[p])

        def wait_h(p):
            pltpu.make_async_copy(
                h_hbm.at[cols_v.at[0]], hrs.at[p], semH[p]).wait()

        def sblock(u, carry):
            off = pl.multiple_of(u * sb, 8)
            usl = pl.ds(off, sb)
            cps = [pltpu.async_copy(rows_hbm.at[wid].at[usl], rows_v, semST),
                   pltpu.async_copy(cols_hbm.at[wid].at[usl], cols_v, semST),
                   pltpu.async_copy(b_hbm.at[wid].at[usl], b_v, semST),
                   pltpu.async_copy(adjv_hbm.at[wid].at[usl], adjv_v, semST)]
            for cp in cps:
                cp.wait()
            fire_h(0, 0)
            fire_h(1, 1)

            def loop_body(it, carry2):
                j = it * 2
                wait_h(0)

                @pl.when(it > 0)
                def _():
                    drain_scatter(0)
                process(j, 0)

                @pl.when(j + 2 < sb)
                def _():
                    fire_h(jnp.where(j + 2 < sb, j + 2, 0), 0)
                wait_h(1)

                @pl.when(it > 0)
                def _():
                    drain_scatter(1)
                process(j + 1, 1)

                @pl.when(j + 3 < sb)
                def _():
                    fire_h(jnp.where(j + 3 < sb, j + 3, 0), 1)
                return carry2

            lax.fori_loop(0, sb // 2, loop_body, 0)
            drain_scatter(0)
            drain_scatter(1)
            return carry

        lax.fori_loop(0, nsb, sblock, 0)

        plsc.subcore_barrier()
        for k in range(kz):
            c = sid * kz + k
            base = c * _CL
            build_idx(base)
            pltpu.sync_copy(acc.at[idx_v], valss.at[0])
            obase = pl.multiple_of(c * _CL, _CL)
            pltpu.sync_copy(valss.at[0], out_hbm.at[cid].at[pl.ds(obase, _CL)])

    fn = pl.kernel(body, out_type=out_type, mesh=mesh,
                   scratch_types=scratch, interpret=interpret,
                   compiler_params=pltpu.CompilerParams(
                       needs_layout_passes=False))
    (acc_out,) = fn(rows3, cols3, b3, adjv3, s, d, m, h)
    return acc_out[:, :n, :]


def _run(node_feats, edge_feats, edge_indices, adj, W0, a_src0, a_dst0, a_e0,
         W1, a_src1, a_dst1, a_e1, interpret=False):
    n = node_feats.shape[0]
    e = edge_feats.shape[0]
    hid = W0.shape[1]
    ncls = W1.shape[1]
    fr0 = hid
    fr1 = ncls
    fpad = 128
    bm = 200 if n % 200 == 0 else n

    rows = edge_indices[0].astype(jnp.int32)
    cols = edge_indices[1].astype(jnp.int32)
    ee0, ee1, cell = _edge_prep(edge_feats, a_e0, a_e1, rows, cols, n,
                                interpret=interpret)
    ept = -(-e // _NW)
    nch = -(-ept // _CL)
    nch += (-nch) % 16
    ep = _NW * nch * _CL

    def to3(x):
        return jnp.pad(x, (0, ep - e)).reshape(_NW, nch, _CL)

    rows3, cols3, cell3 = to3(rows), to3(cols), to3(cell)
    b03, b13 = to3(ee0), to3(ee1)
    adjflat = adj.reshape(-1).astype(jnp.int32)
    adjv3 = _sc_adj_gather(cell3, adjflat, interpret=interpret)

    h0, hp0, s0, d0 = _prep_layer(node_feats, W0, a_src0, a_dst0, fpad,
                                  interpret=interpret)
    m0, den0, num0 = _dense_pass(s0, d0[:, 0], adj, h0, bm,
                                 interpret=interpret)
    acc0 = _sc_corrections(rows3, cols3, b03, adjv3, s0[:, 0], d0[:, 0],
                           m0[:, 0], hp0, fr0, interpret=interpret)
    accs0 = acc0[0] + acc0[1]
    dden0 = accs0[:, fr0]

    h1, hp1, s1, d1 = _combine_prep(num0, accs0, den0, dden0, W1, a_src1,
                                    a_dst1, fpad, interpret=interpret)
    m1, den1, num1 = _dense_pass(s1, d1[:, 0], adj, h1, bm,
                                 interpret=interpret)
    acc1 = _sc_corrections(rows3, cols3, b13, adjv3, s1[:, 0], d1[:, 0],
                           m1[:, 0], hp1, fr1, interpret=interpret)
    accs1 = acc1[0] + acc1[1]
    dden1 = accs1[:, fr1]

    return _final_combine(num1, accs1, den1, dden1, ncls,
                          interpret=interpret)


def kernel(node_feats, edge_feats, edge_indices, adj, W0, a_src0, a_dst0,
           a_e0, W1, a_src1, a_dst1, a_e1):
    return _run(node_feats, edge_feats, edge_indices, adj, W0, a_src0,
                a_dst0, a_e0, W1, a_src1, a_dst1, a_e1)
